# baseline (speedup 1.0000x reference)
import functools

import jax
import jax.numpy as jnp
from jax.experimental import pallas as pl
from jax.experimental.pallas import tpu as pltpu

H = 128
NG = 100
L = 6
IN_DIM = 5
CUTOFF = 10.0
N = 10000
E = 320000
NL = 2000

BN = 2000
BE = 2000

_LOG2 = 0.6931471805599453


def _mm(a, b):
    return jax.lax.dot_general(a, b, (((1,), (1,)), ((), ())),
                               preferred_element_type=jnp.float32)


def _sp(x):
    return jax.nn.softplus(x) - _LOG2


def _enc_body(z_ref, lew1, leb1, lew2, leb2, pew1, peb1, pew2, peb2, l1,
              h_ref, xl_ref):
    z = z_ref[...]
    hl = _mm(jax.nn.silu(_mm(z, lew1[...]) + leb1[...]), lew2[...]) + leb2[...]
    hp = _mm(jax.nn.silu(_mm(z, pew1[...]) + peb1[...]), pew2[...]) + peb2[...]
    rows = pl.program_id(0) * BN + jax.lax.broadcasted_iota(jnp.int32, (BN, 1), 0)
    h = jnp.where(rows < NL, hl, hp)
    h_ref[...] = h
    xl_ref[...] = _mm(h, l1[...])


def _full(shape):
    nd = len(shape)
    return pl.BlockSpec(shape, lambda *_: (0,) * nd)


def _encoder(z, lew1, leb1, lew2, leb2, pew1, peb1, pew2, peb2, l1):
    grid = N // BN
    specs = [pl.BlockSpec((BN, IN_DIM), lambda i: (i, 0))]
    for w in (lew1, leb1, lew2, leb2, pew1, peb1, pew2, peb2, l1):
        specs.append(_full(w.shape))
    return pl.pallas_call(
        _enc_body,
        grid=(grid,),
        in_specs=specs,
        out_specs=[pl.BlockSpec((BN, H), lambda i: (i, 0))] * 2,
        out_shape=[jax.ShapeDtypeStruct((N, H), jnp.float32)] * 2,
    )(z, lew1, leb1, lew2, leb2, pew1, peb1, pew2, peb2, l1)


def _edge_body(first, ea_ref, xlg_ref, rr_ref, rc_ref, cin_ref,
               w1, b1, w2, b2, cw1, cb1, cw2, cb2, cw3, scl,
               mij_ref, ced_ref, cout_ref):
    cd = rr_ref[...] - rc_ref[...]
    radial = jnp.sum(cd * cd, axis=1, keepdims=True)
    if first:
        C = (radial <= CUTOFF).astype(jnp.float32)
        cout_ref[...] = C
    else:
        C = cin_ref[...]
    t = _sp(_mm(ea_ref[...], w1[...]) + b1[...])
    W = (_mm(t, w2[...]) + b2[...]) * C
    mij = xlg_ref[...] * W
    mij_ref[...] = mij
    nrm2 = jnp.sqrt(radial + 1e-8)
    cdn = cd / (nrm2 + 1.0)
    cn = jnp.sqrt(jnp.sum(cdn * cdn, axis=1, keepdims=True))
    cdn = cdn / jnp.maximum(cn, 1e-8) * scl[0, 0]
    t1 = jax.nn.silu(_mm(mij, cw1[...]) + cb1[...])
    t2 = jax.nn.silu(_mm(t1, cw2[...]) + cb2[...])
    ce = jnp.sum(t2 * cw3[...], axis=1, keepdims=True)
    ced_ref[...] = ce * cdn


def _edge_stage(first, ea, xlg, rr4, rc4, cin, w1, b1, w2, b2,
                cw1, cb1, cw2, cb2, cw3, scl):
    grid = E // BE
    especs = [
        pl.BlockSpec((BE, NG), lambda i: (i, 0)),
        pl.BlockSpec((BE, H), lambda i: (i, 0)),
        pl.BlockSpec((BE, 4), lambda i: (i, 0)),
        pl.BlockSpec((BE, 4), lambda i: (i, 0)),
        pl.BlockSpec((BE, 1), lambda i: (i, 0)),
    ]
    for w in (w1, b1, w2, b2, cw1, cb1, cw2, cb2, cw3, scl):
        especs.append(_full(w.shape))
    return pl.pallas_call(
        functools.partial(_edge_body, first),
        grid=(grid,),
        in_specs=especs,
        out_specs=[
            pl.BlockSpec((BE, H), lambda i: (i, 0)),
            pl.BlockSpec((BE, 4), lambda i: (i, 0)),
            pl.BlockSpec((BE, 1), lambda i: (i, 0)),
        ],
        out_shape=[
            jax.ShapeDtypeStruct((E, H), jnp.float32),
            jax.ShapeDtypeStruct((E, 4), jnp.float32),
            jax.ShapeDtypeStruct((E, 1), jnp.float32),
        ],
    )(ea, xlg, rr4, rc4, cin, w1, b1, w2, b2, cw1, cb1, cw2, cb2, cw3, scl)


def _node_body(h_ref, m0_ref, m1_ref, r0_ref, r1_ref, r4_ref,
               l2, l2b, lwa, lwb, lb, l1n,
               hn_ref, xln_ref, r4n_ref):
    m = _mm(m0_ref[...] + m1_ref[...], l2[...]) + l2b[...]
    hn = _mm(h_ref[...], lwa[...]) + _mm(_sp(m), lwb[...]) + lb[...]
    hn_ref[...] = hn
    xln_ref[...] = _mm(hn, l1n[...])
    rows = pl.program_id(0) * BN + jax.lax.broadcasted_iota(jnp.int32, (BN, 1), 0)
    upd = r0_ref[...] + r1_ref[...]
    r4n_ref[...] = r4_ref[...] + jnp.where(rows < NL, upd, 0.0)


def _node_stage(h, m0, m1, r0, r1, r4, l2, l2b, lwa, lwb, lb, l1n):
    grid = N // BN
    specs = [
        pl.BlockSpec((BN, H), lambda i: (i, 0)),
        pl.BlockSpec((BN, H), lambda i: (i, 0)),
        pl.BlockSpec((BN, H), lambda i: (i, 0)),
        pl.BlockSpec((BN, 4), lambda i: (i, 0)),
        pl.BlockSpec((BN, 4), lambda i: (i, 0)),
        pl.BlockSpec((BN, 4), lambda i: (i, 0)),
    ]
    for w in (l2, l2b, lwa, lwb, lb, l1n):
        specs.append(_full(w.shape))
    return pl.pallas_call(
        _node_body,
        grid=(grid,),
        in_specs=specs,
        out_specs=[
            pl.BlockSpec((BN, H), lambda i: (i, 0)),
            pl.BlockSpec((BN, H), lambda i: (i, 0)),
            pl.BlockSpec((BN, 4), lambda i: (i, 0)),
        ],
        out_shape=[
            jax.ShapeDtypeStruct((N, H), jnp.float32),
            jax.ShapeDtypeStruct((N, H), jnp.float32),
            jax.ShapeDtypeStruct((N, 4), jnp.float32),
        ],
    )(h, m0, m1, r0, r1, r4, l2, l2b, lwa, lwb, lb, l1n)


def kernel(z, pos, edge_index, edge_attr, ligand_batch,
           le_w1, le_b1, le_w2, le_b2, pe_w1, pe_b1, pe_w2, pe_b2,
           mlp_w1, mlp_b1, mlp_w2, mlp_b2, lin1_w, lin2_w, lin2_b,
           lin_w, lin_b, cn_w1, cn_b1, cn_w2, cn_b2, cn_w3, scale):
    row = edge_index[0].astype(jnp.int32)
    col = edge_index[1].astype(jnp.int32)
    r4 = jnp.concatenate([pos, jnp.zeros((N, 1), jnp.float32)], axis=1)

    b2d = lambda b: b.reshape(1, -1)
    h, xl = _encoder(z, le_w1, b2d(le_b1), le_w2, b2d(le_b2),
                     pe_w1, b2d(pe_b1), pe_w2, b2d(pe_b2), lin1_w[0])

    zeros_h = jnp.zeros((N, H), jnp.float32)
    zeros_4 = jnp.zeros((N, 4), jnp.float32)
    C = jnp.zeros((E, 1), jnp.float32)

    for i in range(L):
        xlg = jnp.take(xl, row, axis=0)
        rr4 = jnp.take(r4, row, axis=0)
        rc4 = jnp.take(r4, col, axis=0)
        mij, ced, cout = _edge_stage(
            i == 0, edge_attr, xlg, rr4, rc4, C,
            mlp_w1[i], b2d(mlp_b1[i]), mlp_w2[i], b2d(mlp_b2[i]),
            cn_w1[i], b2d(cn_b1[i]), cn_w2[i], b2d(cn_b2[i]),
            cn_w3[i], scale[i].reshape(1, 1))
        if i == 0:
            C = cout
        m_i = jax.ops.segment_sum(mij, col, num_segments=N)
        racc = jax.ops.segment_sum(ced, col, num_segments=N)
        l1n = lin1_w[(i + 1) % L]
        h, xl, r4 = _node_stage(
            h, m_i, zeros_h, racc, zeros_4, r4,
            lin2_w[i], b2d(lin2_b[i]),
            lin_w[i][:, :H], lin_w[i][:, H:], b2d(lin_b[i]), l1n)

    return (h, r4[:, :3])


# speedup vs baseline: 1.5995x; 1.5995x over previous
import functools

import jax
import jax.numpy as jnp
from jax import lax
from jax.experimental import pallas as pl
from jax.experimental.pallas import tpu as pltpu
from jax.experimental.pallas import tpu_sc as plsc

H = 128
NG = 100
L = 6
IN_DIM = 5
CUTOFF = 10.0
N = 10000
E = 320000
NL = 2000

BN = 2000
BE = 2000

_LOG2 = 0.6931471805599453


def _mm(a, b):
    return jax.lax.dot_general(a, b, (((1,), (1,)), ((), ())),
                               preferred_element_type=jnp.float32)


def _sp(x):
    return jax.nn.softplus(x) - _LOG2


NWORK = 32
EPW = E // NWORK
CH = 80
NCH = EPW // CH
SLAB = 640
NSLAB_LAST = N - 15 * SLAB

_sc_mesh = plsc.VectorSubcoreMesh(core_axis_name="c", subcore_axis_name="s",
                                  num_cores=2, num_subcores=16)

def _sc_gather_body(xl_hbm, r4_hbm, row_hbm, col_hbm,
                    xlg_hbm, rr_hbm, rc_hbm,
                    idxr, idxc, xbuf, rrbuf, rcbuf, r4loc, sem):
    cid = lax.axis_index("c")
    sid = lax.axis_index("s")
    base = (sid * 2 + cid) * EPW
    pltpu.sync_copy(r4_hbm, r4loc)
    iota = lax.iota(jnp.int32, 16)
    pos16 = iota // 4
    comp16 = iota % 4

    def chunk(j, _):
        off = pl.multiple_of(base + j * CH, 8)
        pltpu.sync_copy(row_hbm.at[pl.ds(off, CH)], idxr)
        pltpu.sync_copy(col_hbm.at[pl.ds(off, CH)], idxc)
        pltpu.async_copy(xl_hbm.at[idxr], xbuf, sem).wait()
        pltpu.sync_copy(xbuf, xlg_hbm.at[pl.ds(off, CH)])
        for i in range(CH * 4 // 16):
            pv = i * 4 + pos16
            er = plsc.load_gather(idxr, [pv])
            rrbuf[pl.ds(i * 16, 16)] = plsc.load_gather(r4loc, [er * 4 + comp16])
            ec = plsc.load_gather(idxc, [pv])
            rcbuf[pl.ds(i * 16, 16)] = plsc.load_gather(r4loc, [ec * 4 + comp16])
        off4 = pl.multiple_of(off * 4, 8)
        pltpu.sync_copy(rrbuf, rr_hbm.at[pl.ds(off4, CH * 4)])
        pltpu.sync_copy(rcbuf, rc_hbm.at[pl.ds(off4, CH * 4)])
        return _

    lax.fori_loop(0, NCH, chunk, None)


def _sc_gather(xl, r4flat, row, col):
    f = pl.kernel(
        _sc_gather_body,
        out_type=[jax.ShapeDtypeStruct((E, H), jnp.float32),
                  jax.ShapeDtypeStruct((E * 4,), jnp.float32),
                  jax.ShapeDtypeStruct((E * 4,), jnp.float32)],
        mesh=_sc_mesh,
        scratch_types=[
            pltpu.VMEM((CH,), jnp.int32),
            pltpu.VMEM((CH,), jnp.int32),
            pltpu.VMEM((CH, H), jnp.float32),
            pltpu.VMEM((CH * 4,), jnp.float32),
            pltpu.VMEM((CH * 4,), jnp.float32),
            pltpu.VMEM((N * 4,), jnp.float32),
            pltpu.SemaphoreType.DMA,
        ],
        compiler_params=pltpu.CompilerParams(needs_layout_passes=False),
    )
    return f(xl, r4flat, row, col)


def _sc_scatter_body(mij_hbm, ced_hbm, col_hbm, z128_hbm, z4_hbm,
                     macc_hbm, racc_hbm,
                     cidx, mbuf, cbuf, accm, accr, sem):
    cid = lax.axis_index("c")
    sid = lax.axis_index("s")
    base = (sid * 2 + cid) * EPW

    def slab_io(fn):
        @pl.when(sid < 15)
        def _():
            off = pl.multiple_of(sid * SLAB, 8)
            fn(pl.ds(off, SLAB))

        @pl.when(sid == 15)
        def _():
            fn(pl.ds(15 * SLAB, NSLAB_LAST))

    def init(slab):
        pltpu.sync_copy(z128_hbm.at[slab], accm.at[slab])
        pltpu.sync_copy(z4_hbm.at[slab], accr.at[slab])

    slab_io(init)
    plsc.subcore_barrier()

    def chunk(j, _):
        off = pl.multiple_of(base + j * CH, 8)
        pltpu.sync_copy(col_hbm.at[pl.ds(off, CH)], cidx)
        pltpu.sync_copy(mij_hbm.at[pl.ds(off, CH)], mbuf)
        pltpu.sync_copy(ced_hbm.at[pl.ds(off, CH)], cbuf)
        pltpu.sync_copy(mbuf, accm.at[cidx], add=True)
        pltpu.sync_copy(cbuf, accr.at[cidx], add=True)
        return _

    lax.fori_loop(0, NCH, chunk, None)
    plsc.subcore_barrier()

    def flush(slab):
        pltpu.sync_copy(accm.at[slab], macc_hbm.at[cid, slab])
        pltpu.sync_copy(accr.at[slab], racc_hbm.at[cid, slab])

    slab_io(flush)


def _sc_scatter(mij, ced4, col, z128, z4):
    f = pl.kernel(
        _sc_scatter_body,
        out_type=[jax.ShapeDtypeStruct((2, N, H), jnp.float32),
                  jax.ShapeDtypeStruct((2, N, 4), jnp.float32)],
        mesh=_sc_mesh,
        scratch_types=[
            pltpu.VMEM((CH,), jnp.int32),
            pltpu.VMEM((CH, H), jnp.float32),
            pltpu.VMEM((CH, 4), jnp.float32),
            pltpu.VMEM_SHARED((N, H), jnp.float32),
            pltpu.VMEM_SHARED((N, 4), jnp.float32),
            pltpu.SemaphoreType.DMA,
        ],
    )
    return f(mij, ced4, col, z128, z4)


def _enc_body(z_ref, lew1, leb1, lew2, leb2, pew1, peb1, pew2, peb2, l1,
              h_ref, xl_ref):
    z = z_ref[...]
    hl = _mm(jax.nn.silu(_mm(z, lew1[...]) + leb1[...]), lew2[...]) + leb2[...]
    hp = _mm(jax.nn.silu(_mm(z, pew1[...]) + peb1[...]), pew2[...]) + peb2[...]
    rows = pl.program_id(0) * BN + jax.lax.broadcasted_iota(jnp.int32, (BN, 1), 0)
    h = jnp.where(rows < NL, hl, hp)
    h_ref[...] = h
    xl_ref[...] = _mm(h, l1[...])


def _full(shape):
    nd = len(shape)
    return pl.BlockSpec(shape, lambda *_: (0,) * nd)


def _encoder(z, lew1, leb1, lew2, leb2, pew1, peb1, pew2, peb2, l1):
    grid = N // BN
    specs = [pl.BlockSpec((BN, IN_DIM), lambda i: (i, 0))]
    for w in (lew1, leb1, lew2, leb2, pew1, peb1, pew2, peb2, l1):
        specs.append(_full(w.shape))
    return pl.pallas_call(
        _enc_body,
        grid=(grid,),
        in_specs=specs,
        out_specs=[pl.BlockSpec((BN, H), lambda i: (i, 0))] * 2,
        out_shape=[jax.ShapeDtypeStruct((N, H), jnp.float32)] * 2,
    )(z, lew1, leb1, lew2, leb2, pew1, peb1, pew2, peb2, l1)


def _edge_body(first, ea_ref, xlg_ref, rr_ref, rc_ref, cin_ref,
               w1, b1, w2, b2, cw1, cb1, cw2, cb2, cw3, scl,
               mij_ref, ced_ref, cout_ref):
    cd = rr_ref[...] - rc_ref[...]
    radial = jnp.sum(cd * cd, axis=1, keepdims=True)
    if first:
        C = (radial <= CUTOFF).astype(jnp.float32)
        cout_ref[...] = C
    else:
        C = cin_ref[...]
    t = _sp(_mm(ea_ref[...], w1[...]) + b1[...])
    W = (_mm(t, w2[...]) + b2[...]) * C
    mij = xlg_ref[...] * W
    mij_ref[...] = mij
    nrm2 = jnp.sqrt(radial + 1e-8)
    cdn = cd / (nrm2 + 1.0)
    cn = jnp.sqrt(jnp.sum(cdn * cdn, axis=1, keepdims=True))
    cdn = cdn / jnp.maximum(cn, 1e-8) * scl[0, 0]
    t1 = jax.nn.silu(_mm(mij, cw1[...]) + cb1[...])
    t2 = jax.nn.silu(_mm(t1, cw2[...]) + cb2[...])
    ce = jnp.sum(t2 * cw3[...], axis=1, keepdims=True)
    ced_ref[...] = ce * cdn


def _edge_stage(first, ea, xlg, rr4, rc4, cin, w1, b1, w2, b2,
                cw1, cb1, cw2, cb2, cw3, scl):
    grid = E // BE
    especs = [
        pl.BlockSpec((BE, NG), lambda i: (i, 0)),
        pl.BlockSpec((BE, H), lambda i: (i, 0)),
        pl.BlockSpec((BE, 4), lambda i: (i, 0)),
        pl.BlockSpec((BE, 4), lambda i: (i, 0)),
        pl.BlockSpec((BE, 1), lambda i: (i, 0)),
    ]
    for w in (w1, b1, w2, b2, cw1, cb1, cw2, cb2, cw3, scl):
        especs.append(_full(w.shape))
    return pl.pallas_call(
        functools.partial(_edge_body, first),
        grid=(grid,),
        in_specs=especs,
        out_specs=[
            pl.BlockSpec((BE, H), lambda i: (i, 0)),
            pl.BlockSpec((BE, 4), lambda i: (i, 0)),
            pl.BlockSpec((BE, 1), lambda i: (i, 0)),
        ],
        out_shape=[
            jax.ShapeDtypeStruct((E, H), jnp.float32),
            jax.ShapeDtypeStruct((E, 4), jnp.float32),
            jax.ShapeDtypeStruct((E, 1), jnp.float32),
        ],
    )(ea, xlg, rr4, rc4, cin, w1, b1, w2, b2, cw1, cb1, cw2, cb2, cw3, scl)


def _node_body(h_ref, m0_ref, m1_ref, r0_ref, r1_ref, r4_ref,
               l2, l2b, lwa, lwb, lb, l1n,
               hn_ref, xln_ref, r4n_ref):
    m = _mm(m0_ref[...] + m1_ref[...], l2[...]) + l2b[...]
    hn = _mm(h_ref[...], lwa[...]) + _mm(_sp(m), lwb[...]) + lb[...]
    hn_ref[...] = hn
    xln_ref[...] = _mm(hn, l1n[...])
    rows = pl.program_id(0) * BN + jax.lax.broadcasted_iota(jnp.int32, (BN, 1), 0)
    upd = r0_ref[...] + r1_ref[...]
    r4n_ref[...] = r4_ref[...] + jnp.where(rows < NL, upd, 0.0)


def _node_stage(h, m0, m1, r0, r1, r4, l2, l2b, lwa, lwb, lb, l1n):
    grid = N // BN
    specs = [
        pl.BlockSpec((BN, H), lambda i: (i, 0)),
        pl.BlockSpec((BN, H), lambda i: (i, 0)),
        pl.BlockSpec((BN, H), lambda i: (i, 0)),
        pl.BlockSpec((BN, 4), lambda i: (i, 0)),
        pl.BlockSpec((BN, 4), lambda i: (i, 0)),
        pl.BlockSpec((BN, 4), lambda i: (i, 0)),
    ]
    for w in (l2, l2b, lwa, lwb, lb, l1n):
        specs.append(_full(w.shape))
    return pl.pallas_call(
        _node_body,
        grid=(grid,),
        in_specs=specs,
        out_specs=[
            pl.BlockSpec((BN, H), lambda i: (i, 0)),
            pl.BlockSpec((BN, H), lambda i: (i, 0)),
            pl.BlockSpec((BN, 4), lambda i: (i, 0)),
        ],
        out_shape=[
            jax.ShapeDtypeStruct((N, H), jnp.float32),
            jax.ShapeDtypeStruct((N, H), jnp.float32),
            jax.ShapeDtypeStruct((N, 4), jnp.float32),
        ],
    )(h, m0, m1, r0, r1, r4, l2, l2b, lwa, lwb, lb, l1n)


def kernel(z, pos, edge_index, edge_attr, ligand_batch,
           le_w1, le_b1, le_w2, le_b2, pe_w1, pe_b1, pe_w2, pe_b2,
           mlp_w1, mlp_b1, mlp_w2, mlp_b2, lin1_w, lin2_w, lin2_b,
           lin_w, lin_b, cn_w1, cn_b1, cn_w2, cn_b2, cn_w3, scale):
    row = edge_index[0].astype(jnp.int32)
    col = edge_index[1].astype(jnp.int32)
    r4 = jnp.concatenate([pos, jnp.zeros((N, 1), jnp.float32)], axis=1)

    b2d = lambda b: b.reshape(1, -1)
    h, xl = _encoder(z, le_w1, b2d(le_b1), le_w2, b2d(le_b2),
                     pe_w1, b2d(pe_b1), pe_w2, b2d(pe_b2), lin1_w[0])

    zeros_h = jnp.zeros((N, H), jnp.float32)
    zeros_4 = jnp.zeros((N, 4), jnp.float32)
    C = jnp.zeros((E, 1), jnp.float32)

    for i in range(L):
        xlg, rrflat, rcflat = _sc_gather(xl, r4.reshape(-1), row, col)
        rr4 = rrflat.reshape(E, 4)
        rc4 = rcflat.reshape(E, 4)
        mij, ced, cout = _edge_stage(
            i == 0, edge_attr, xlg, rr4, rc4, C,
            mlp_w1[i], b2d(mlp_b1[i]), mlp_w2[i], b2d(mlp_b2[i]),
            cn_w1[i], b2d(cn_b1[i]), cn_w2[i], b2d(cn_b2[i]),
            cn_w3[i], scale[i].reshape(1, 1))
        if i == 0:
            C = cout
        m_i = jax.ops.segment_sum(mij, col, num_segments=N)
        racc1 = jax.ops.segment_sum(ced, col, num_segments=N)
        l1n = lin1_w[(i + 1) % L]
        h, xl, r4 = _node_stage(
            h, m_i, zeros_h, racc1, zeros_4, r4,
            lin2_w[i], b2d(lin2_b[i]),
            lin_w[i][:, :H], lin_w[i][:, H:], b2d(lin_b[i]), l1n)

    return (h, r4[:, :3])


# speedup vs baseline: 1.9279x; 1.2053x over previous
import functools

import jax
import jax.numpy as jnp
from jax import lax
from jax.experimental import pallas as pl
from jax.experimental.pallas import tpu as pltpu
from jax.experimental.pallas import tpu_sc as plsc

H = 128
NG = 100
L = 6
IN_DIM = 5
CUTOFF = 10.0
N = 10000
E = 320000
NL = 2000

BN = 2000
BE = 2000

_LOG2 = 0.6931471805599453


def _mm(a, b):
    return jax.lax.dot_general(a, b, (((1,), (1,)), ((), ())),
                               preferred_element_type=jnp.float32)


def _sp(x):
    return jax.nn.softplus(x) - _LOG2


NWORK = 32
EPW = E // NWORK
CH = 80
NCH = EPW // CH
SLAB = 640
NSLAB_LAST = N - 15 * SLAB

_sc_mesh = plsc.VectorSubcoreMesh(core_axis_name="c", subcore_axis_name="s",
                                  num_cores=2, num_subcores=16)

def _sc_gather_body(xl_hbm, r4_hbm, row_hbm, col_hbm,
                    xlg_hbm, rr_hbm, rc_hbm,
                    idxr, idxc, xbuf, rrbuf, rcbuf, r4loc, sem):
    cid = lax.axis_index("c")
    sid = lax.axis_index("s")
    base = (sid * 2 + cid) * EPW
    pltpu.sync_copy(r4_hbm, r4loc)
    iota = lax.iota(jnp.int32, 16)
    pos16 = iota // 4
    comp16 = iota % 4

    def chunk(j, _):
        off = pl.multiple_of(base + j * CH, 8)
        pltpu.sync_copy(row_hbm.at[pl.ds(off, CH)], idxr)
        pltpu.sync_copy(col_hbm.at[pl.ds(off, CH)], idxc)
        pltpu.async_copy(xl_hbm.at[idxr], xbuf, sem).wait()
        pltpu.sync_copy(xbuf, xlg_hbm.at[pl.ds(off, CH)])
        for i in range(CH * 4 // 16):
            pv = i * 4 + pos16
            er = plsc.load_gather(idxr, [pv])
            rrbuf[pl.ds(i * 16, 16)] = plsc.load_gather(r4loc, [er * 4 + comp16])
            ec = plsc.load_gather(idxc, [pv])
            rcbuf[pl.ds(i * 16, 16)] = plsc.load_gather(r4loc, [ec * 4 + comp16])
        off4 = pl.multiple_of(off * 4, 8)
        pltpu.sync_copy(rrbuf, rr_hbm.at[pl.ds(off4, CH * 4)])
        pltpu.sync_copy(rcbuf, rc_hbm.at[pl.ds(off4, CH * 4)])
        return _

    lax.fori_loop(0, NCH, chunk, None)


def _sc_gather(xl, r4flat, row, col):
    f = pl.kernel(
        _sc_gather_body,
        out_type=[jax.ShapeDtypeStruct((E, H), jnp.float32),
                  jax.ShapeDtypeStruct((E * 4,), jnp.float32),
                  jax.ShapeDtypeStruct((E * 4,), jnp.float32)],
        mesh=_sc_mesh,
        scratch_types=[
            pltpu.VMEM((CH,), jnp.int32),
            pltpu.VMEM((CH,), jnp.int32),
            pltpu.VMEM((CH, H), jnp.float32),
            pltpu.VMEM((CH * 4,), jnp.float32),
            pltpu.VMEM((CH * 4,), jnp.float32),
            pltpu.VMEM((N * 4,), jnp.float32),
            pltpu.SemaphoreType.DMA,
        ],
        compiler_params=pltpu.CompilerParams(needs_layout_passes=False),
    )
    return f(xl, r4flat, row, col)


def _sc_scatter_body(mij_hbm, col_hbm, z128_hbm,
                     macc_hbm,
                     cidx, mbuf, accm, sem):
    cid = lax.axis_index("c")
    sid = lax.axis_index("s")
    base = (sid * 2 + cid) * EPW

    def slab_io(fn):
        @pl.when(sid < 15)
        def _():
            off = pl.multiple_of(sid * SLAB, 8)
            fn(pl.ds(off, SLAB))

        @pl.when(sid == 15)
        def _():
            fn(pl.ds(15 * SLAB, NSLAB_LAST))

    def init(slab):
        pltpu.sync_copy(z128_hbm.at[slab], accm.at[slab])

    slab_io(init)
    plsc.subcore_barrier()

    def chunk(j, _):
        off = pl.multiple_of(base + j * CH, 8)
        pltpu.sync_copy(col_hbm.at[pl.ds(off, CH)], cidx)
        pltpu.sync_copy(mij_hbm.at[pl.ds(off, CH)], mbuf)
        pltpu.sync_copy(mbuf, accm.at[cidx], add=True)
        return _

    lax.fori_loop(0, NCH, chunk, None)
    plsc.subcore_barrier()

    def flush(slab):
        pltpu.sync_copy(accm.at[slab], macc_hbm.at[cid, slab])

    slab_io(flush)


def _sc_scatter(mij, col, z128):
    f = pl.kernel(
        _sc_scatter_body,
        out_type=[jax.ShapeDtypeStruct((2, N, H), jnp.float32)],
        mesh=_sc_mesh,
        scratch_types=[
            pltpu.VMEM((CH,), jnp.int32),
            pltpu.VMEM((CH, H), jnp.float32),
            pltpu.VMEM_SHARED((N, H), jnp.float32),
            pltpu.SemaphoreType.DMA,
        ],
    )
    return f(mij, col, z128)


def _enc_body(z_ref, lew1, leb1, lew2, leb2, pew1, peb1, pew2, peb2, l1,
              h_ref, xl_ref):
    z = z_ref[...]
    hl = _mm(jax.nn.silu(_mm(z, lew1[...]) + leb1[...]), lew2[...]) + leb2[...]
    hp = _mm(jax.nn.silu(_mm(z, pew1[...]) + peb1[...]), pew2[...]) + peb2[...]
    rows = pl.program_id(0) * BN + jax.lax.broadcasted_iota(jnp.int32, (BN, 1), 0)
    h = jnp.where(rows < NL, hl, hp)
    h_ref[...] = h
    xl_ref[...] = _mm(h, l1[...])


def _full(shape):
    nd = len(shape)
    return pl.BlockSpec(shape, lambda *_: (0,) * nd)


def _encoder(z, lew1, leb1, lew2, leb2, pew1, peb1, pew2, peb2, l1):
    grid = N // BN
    specs = [pl.BlockSpec((BN, IN_DIM), lambda i: (i, 0))]
    for w in (lew1, leb1, lew2, leb2, pew1, peb1, pew2, peb2, l1):
        specs.append(_full(w.shape))
    return pl.pallas_call(
        _enc_body,
        grid=(grid,),
        in_specs=specs,
        out_specs=[pl.BlockSpec((BN, H), lambda i: (i, 0))] * 2,
        out_shape=[jax.ShapeDtypeStruct((N, H), jnp.float32)] * 2,
    )(z, lew1, leb1, lew2, leb2, pew1, peb1, pew2, peb2, l1)


def _edge_body(first, ea_ref, xlg_ref, rr_ref, rc_ref, cin_ref,
               w1, b1, w2, b2, cw1, cb1, cw2, cb2, cw3, scl,
               mij_ref, ced_ref, cout_ref):
    cd = rr_ref[...] - rc_ref[...]
    radial = jnp.sum(cd * cd, axis=1, keepdims=True)
    if first:
        C = (radial <= CUTOFF).astype(jnp.float32)
        cout_ref[...] = C
    else:
        C = cin_ref[...]
    t = _sp(_mm(ea_ref[...], w1[...]) + b1[...])
    W = (_mm(t, w2[...]) + b2[...]) * C
    mij = xlg_ref[...] * W
    mij_ref[...] = mij
    nrm2 = jnp.sqrt(radial + 1e-8)
    cdn = cd / (nrm2 + 1.0)
    cn = jnp.sqrt(jnp.sum(cdn * cdn, axis=1, keepdims=True))
    cdn = cdn / jnp.maximum(cn, 1e-8) * scl[0, 0]
    t1 = jax.nn.silu(_mm(mij, cw1[...]) + cb1[...])
    t2 = jax.nn.silu(_mm(t1, cw2[...]) + cb2[...])
    ce = jnp.sum(t2 * cw3[...], axis=1, keepdims=True)
    ced_ref[...] = ce * cdn


def _edge_stage(first, ea, xlg, rr4, rc4, cin, w1, b1, w2, b2,
                cw1, cb1, cw2, cb2, cw3, scl):
    grid = E // BE
    especs = [
        pl.BlockSpec((BE, NG), lambda i: (i, 0)),
        pl.BlockSpec((BE, H), lambda i: (i, 0)),
        pl.BlockSpec((BE, 4), lambda i: (i, 0)),
        pl.BlockSpec((BE, 4), lambda i: (i, 0)),
        pl.BlockSpec((BE, 1), lambda i: (i, 0)),
    ]
    for w in (w1, b1, w2, b2, cw1, cb1, cw2, cb2, cw3, scl):
        especs.append(_full(w.shape))
    return pl.pallas_call(
        functools.partial(_edge_body, first),
        grid=(grid,),
        in_specs=especs,
        out_specs=[
            pl.BlockSpec((BE, H), lambda i: (i, 0)),
            pl.BlockSpec((BE, 4), lambda i: (i, 0)),
            pl.BlockSpec((BE, 1), lambda i: (i, 0)),
        ],
        out_shape=[
            jax.ShapeDtypeStruct((E, H), jnp.float32),
            jax.ShapeDtypeStruct((E, 4), jnp.float32),
            jax.ShapeDtypeStruct((E, 1), jnp.float32),
        ],
    )(ea, xlg, rr4, rc4, cin, w1, b1, w2, b2, cw1, cb1, cw2, cb2, cw3, scl)


def _node_body(h_ref, m0_ref, m1_ref, r0_ref, r1_ref, r4_ref,
               l2, l2b, lwa, lwb, lb, l1n,
               hn_ref, xln_ref, r4n_ref):
    m = _mm(m0_ref[...] + m1_ref[...], l2[...]) + l2b[...]
    hn = _mm(h_ref[...], lwa[...]) + _mm(_sp(m), lwb[...]) + lb[...]
    hn_ref[...] = hn
    xln_ref[...] = _mm(hn, l1n[...])
    rows = pl.program_id(0) * BN + jax.lax.broadcasted_iota(jnp.int32, (BN, 1), 0)
    upd = r0_ref[...] + r1_ref[...]
    r4n_ref[...] = r4_ref[...] + jnp.where(rows < NL, upd, 0.0)


def _node_stage(h, m0, m1, r0, r1, r4, l2, l2b, lwa, lwb, lb, l1n):
    grid = N // BN
    specs = [
        pl.BlockSpec((BN, H), lambda i: (i, 0)),
        pl.BlockSpec((BN, H), lambda i: (i, 0)),
        pl.BlockSpec((BN, H), lambda i: (i, 0)),
        pl.BlockSpec((BN, 4), lambda i: (i, 0)),
        pl.BlockSpec((BN, 4), lambda i: (i, 0)),
        pl.BlockSpec((BN, 4), lambda i: (i, 0)),
    ]
    for w in (l2, l2b, lwa, lwb, lb, l1n):
        specs.append(_full(w.shape))
    return pl.pallas_call(
        _node_body,
        grid=(grid,),
        in_specs=specs,
        out_specs=[
            pl.BlockSpec((BN, H), lambda i: (i, 0)),
            pl.BlockSpec((BN, H), lambda i: (i, 0)),
            pl.BlockSpec((BN, 4), lambda i: (i, 0)),
        ],
        out_shape=[
            jax.ShapeDtypeStruct((N, H), jnp.float32),
            jax.ShapeDtypeStruct((N, H), jnp.float32),
            jax.ShapeDtypeStruct((N, 4), jnp.float32),
        ],
    )(h, m0, m1, r0, r1, r4, l2, l2b, lwa, lwb, lb, l1n)


def kernel(z, pos, edge_index, edge_attr, ligand_batch,
           le_w1, le_b1, le_w2, le_b2, pe_w1, pe_b1, pe_w2, pe_b2,
           mlp_w1, mlp_b1, mlp_w2, mlp_b2, lin1_w, lin2_w, lin2_b,
           lin_w, lin_b, cn_w1, cn_b1, cn_w2, cn_b2, cn_w3, scale):
    row = edge_index[0].astype(jnp.int32)
    col = edge_index[1].astype(jnp.int32)
    r4 = jnp.concatenate([pos, jnp.zeros((N, 1), jnp.float32)], axis=1)

    b2d = lambda b: b.reshape(1, -1)
    h, xl = _encoder(z, le_w1, b2d(le_b1), le_w2, b2d(le_b2),
                     pe_w1, b2d(pe_b1), pe_w2, b2d(pe_b2), lin1_w[0])

    zeros_h = jnp.zeros((N, H), jnp.float32)
    zeros_4 = jnp.zeros((N, 4), jnp.float32)
    C = jnp.zeros((E, 1), jnp.float32)

    for i in range(L):
        xlg, rrflat, rcflat = _sc_gather(xl, r4.reshape(-1), row, col)
        rr4 = rrflat.reshape(E, 4)
        rc4 = rcflat.reshape(E, 4)
        mij, ced, cout = _edge_stage(
            i == 0, edge_attr, xlg, rr4, rc4, C,
            mlp_w1[i], b2d(mlp_b1[i]), mlp_w2[i], b2d(mlp_b2[i]),
            cn_w1[i], b2d(cn_b1[i]), cn_w2[i], b2d(cn_b2[i]),
            cn_w3[i], scale[i].reshape(1, 1))
        if i == 0:
            C = cout
        macc, = _sc_scatter(mij, col, zeros_h)
        racc1 = jax.ops.segment_sum(ced, col, num_segments=N)
        l1n = lin1_w[(i + 1) % L]
        h, xl, r4 = _node_stage(
            h, macc[0], macc[1], racc1, zeros_4, r4,
            lin2_w[i], b2d(lin2_b[i]),
            lin_w[i][:, :H], lin_w[i][:, H:], b2d(lin_b[i]), l1n)

    return (h, r4[:, :3])


# speedup vs baseline: 2.3025x; 1.1943x over previous
import functools

import jax
import jax.numpy as jnp
from jax import lax
from jax.experimental import pallas as pl
from jax.experimental.pallas import tpu as pltpu
from jax.experimental.pallas import tpu_sc as plsc

H = 128
NG = 100
L = 6
IN_DIM = 5
CUTOFF = 10.0
N = 10000
E = 320000
NL = 2000

BN = 2000
BE = 2000

_LOG2 = 0.6931471805599453


def _mm(a, b):
    return jax.lax.dot_general(a, b, (((1,), (1,)), ((), ())),
                               preferred_element_type=jnp.float32)


def _sp(x):
    return jax.nn.softplus(x) - _LOG2


NWORK = 32
EPW = E // NWORK
CH = 80
NCH = EPW // CH
SLAB = 640
NSLAB_LAST = N - 15 * SLAB

_sc_mesh = plsc.VectorSubcoreMesh(core_axis_name="c", subcore_axis_name="s",
                                  num_cores=2, num_subcores=16)

def _sc_gather_body(xl_hbm, r4_hbm, row_hbm, col_hbm,
                    xlg_hbm, rr_hbm, rc_hbm,
                    idxr, idxc, xbuf, rrbuf, rcbuf, r4loc, sem):
    cid = lax.axis_index("c")
    sid = lax.axis_index("s")
    base = (sid * 2 + cid) * EPW
    pltpu.sync_copy(r4_hbm, r4loc)
    iota = lax.iota(jnp.int32, 16)
    pos16 = iota // 4
    comp16 = iota % 4

    def chunk(j, _):
        off = pl.multiple_of(base + j * CH, 8)
        pltpu.sync_copy(row_hbm.at[pl.ds(off, CH)], idxr)
        pltpu.sync_copy(col_hbm.at[pl.ds(off, CH)], idxc)
        pltpu.async_copy(xl_hbm.at[idxr], xbuf, sem).wait()
        pltpu.sync_copy(xbuf, xlg_hbm.at[pl.ds(off, CH)])
        for i in range(CH * 4 // 16):
            pv = i * 4 + pos16
            er = plsc.load_gather(idxr, [pv])
            rrbuf[pl.ds(i * 16, 16)] = plsc.load_gather(r4loc, [er * 4 + comp16])
            ec = plsc.load_gather(idxc, [pv])
            rcbuf[pl.ds(i * 16, 16)] = plsc.load_gather(r4loc, [ec * 4 + comp16])
        off4 = pl.multiple_of(off * 4, 8)
        pltpu.sync_copy(rrbuf, rr_hbm.at[pl.ds(off4, CH * 4)])
        pltpu.sync_copy(rcbuf, rc_hbm.at[pl.ds(off4, CH * 4)])
        return _

    lax.fori_loop(0, NCH, chunk, None)


def _sc_gather(xl, r4flat, row, col):
    f = pl.kernel(
        _sc_gather_body,
        out_type=[jax.ShapeDtypeStruct((E, H), jnp.float32),
                  jax.ShapeDtypeStruct((E * 4,), jnp.float32),
                  jax.ShapeDtypeStruct((E * 4,), jnp.float32)],
        mesh=_sc_mesh,
        scratch_types=[
            pltpu.VMEM((CH,), jnp.int32),
            pltpu.VMEM((CH,), jnp.int32),
            pltpu.VMEM((CH, H), jnp.float32),
            pltpu.VMEM((CH * 4,), jnp.float32),
            pltpu.VMEM((CH * 4,), jnp.float32),
            pltpu.VMEM((N * 4,), jnp.float32),
            pltpu.SemaphoreType.DMA,
        ],
        compiler_params=pltpu.CompilerParams(needs_layout_passes=False),
    )
    return f(xl, r4flat, row, col)


RROWS = 320


def _sc_scatter_body(mij_hbm, col_hbm, z128_hbm,
                     macc_hbm,
                     cidx, mbuf, accm, sem):
    cid = lax.axis_index("c")
    sid = lax.axis_index("s")
    base = (sid * 2 + cid) * EPW

    def slab_io(fn):
        @pl.when(sid < 15)
        def _():
            off = pl.multiple_of(sid * SLAB, 8)
            fn(pl.ds(off, SLAB))

        @pl.when(sid == 15)
        def _():
            fn(pl.ds(15 * SLAB, NSLAB_LAST))

    def init(slab):
        pltpu.sync_copy(z128_hbm.at[slab], accm.at[slab])

    slab_io(init)
    plsc.subcore_barrier()

    def chunk(j, _):
        off = pl.multiple_of(base + j * CH, 8)
        pltpu.sync_copy(col_hbm.at[pl.ds(off, CH)], cidx)
        pltpu.sync_copy(mij_hbm.at[pl.ds(off, CH)], mbuf)
        pltpu.sync_copy(mbuf, accm.at[cidx], add=True)
        return _

    lax.fori_loop(0, NCH, chunk, None)
    plsc.subcore_barrier()

    def flush(slab):
        pltpu.sync_copy(accm.at[slab], macc_hbm.at[cid, slab])

    slab_io(flush)


def _sc_scatter(mij, col, z128):
    f = pl.kernel(
        _sc_scatter_body,
        out_type=[jax.ShapeDtypeStruct((2, N, H), jnp.float32)],
        mesh=_sc_mesh,
        scratch_types=[
            pltpu.VMEM((CH,), jnp.int32),
            pltpu.VMEM((CH, H), jnp.float32),
            pltpu.VMEM_SHARED((N, H), jnp.float32),
            pltpu.SemaphoreType.DMA,
        ],
    )
    return f(mij, col, z128)


def _sc_rscatter_body(ced_hbm, col_hbm, z128_hbm,
                      racc_hbm,
                      cidx, cbuf, idxbuf, accr_sh, accr, sem):
    cid = lax.axis_index("c")
    sid = lax.axis_index("s")
    base = (sid * 2 + cid) * EPW
    iota = lax.iota(jnp.int32, 16)
    pos16 = iota // 4
    comp16 = iota % 4

    pltpu.sync_copy(z128_hbm.at[pl.ds(0, RROWS)], accr)

    @pl.when(sid == 0)
    def _():
        pltpu.sync_copy(z128_hbm.at[pl.ds(0, RROWS)], accr_sh)

    plsc.subcore_barrier()

    def chunk(j, _):
        off = pl.multiple_of(base + j * CH, 8)
        pltpu.sync_copy(col_hbm.at[pl.ds(off, CH)], cidx)
        off4 = pl.multiple_of(off * 4, 8)
        pltpu.sync_copy(ced_hbm.at[pl.ds(off4, CH * 4)], cbuf)
        for i in range(CH * 4 // 16):
            pv = i * 4 + pos16
            ec = plsc.load_gather(cidx, [pv])
            a = ec * 4 + comp16
            plsc.addupdate_scatter(accr, [a // 128, a % 128],
                                   cbuf[pl.ds(i * 16, 16)])
        return _

    lax.fori_loop(0, NCH, chunk, None)
    for c3 in range(RROWS // CH):
        for k in range(CH // 16):
            idxbuf[pl.ds(k * 16, 16)] = c3 * CH + k * 16 + iota
        pltpu.sync_copy(accr.at[pl.ds(c3 * CH, CH)], accr_sh.at[idxbuf],
                        add=True)
    plsc.subcore_barrier()

    @pl.when(sid < 2)
    def _():
        half = pl.multiple_of(sid * (RROWS // 2), 8)
        pltpu.sync_copy(accr_sh.at[pl.ds(half, RROWS // 2)],
                        racc_hbm.at[cid, pl.ds(half, RROWS // 2)])


def _sc_rscatter(cedflat, col, z128):
    f = pl.kernel(
        _sc_rscatter_body,
        out_type=[jax.ShapeDtypeStruct((2, RROWS, H), jnp.float32)],
        mesh=_sc_mesh,
        scratch_types=[
            pltpu.VMEM((CH,), jnp.int32),
            pltpu.VMEM((CH * 4,), jnp.float32),
            pltpu.VMEM((CH,), jnp.int32),
            pltpu.VMEM_SHARED((RROWS, H), jnp.float32),
            pltpu.VMEM((RROWS, H), jnp.float32),
            pltpu.SemaphoreType.DMA,
        ],
        compiler_params=pltpu.CompilerParams(needs_layout_passes=False),
    )
    return f(cedflat, col, z128)


def _enc_body(z_ref, lew1, leb1, lew2, leb2, pew1, peb1, pew2, peb2, l1,
              h_ref, xl_ref):
    z = z_ref[...]
    hl = _mm(jax.nn.silu(_mm(z, lew1[...]) + leb1[...]), lew2[...]) + leb2[...]
    hp = _mm(jax.nn.silu(_mm(z, pew1[...]) + peb1[...]), pew2[...]) + peb2[...]
    rows = pl.program_id(0) * BN + jax.lax.broadcasted_iota(jnp.int32, (BN, 1), 0)
    h = jnp.where(rows < NL, hl, hp)
    h_ref[...] = h
    xl_ref[...] = _mm(h, l1[...])


def _full(shape):
    nd = len(shape)
    return pl.BlockSpec(shape, lambda *_: (0,) * nd)


def _encoder(z, lew1, leb1, lew2, leb2, pew1, peb1, pew2, peb2, l1):
    grid = N // BN
    specs = [pl.BlockSpec((BN, IN_DIM), lambda i: (i, 0))]
    for w in (lew1, leb1, lew2, leb2, pew1, peb1, pew2, peb2, l1):
        specs.append(_full(w.shape))
    return pl.pallas_call(
        _enc_body,
        grid=(grid,),
        in_specs=specs,
        out_specs=[pl.BlockSpec((BN, H), lambda i: (i, 0))] * 2,
        out_shape=[jax.ShapeDtypeStruct((N, H), jnp.float32)] * 2,
    )(z, lew1, leb1, lew2, leb2, pew1, peb1, pew2, peb2, l1)


def _edge_body(first, ea_ref, xlg_ref, rr_ref, rc_ref, cin_ref,
               w1, b1, w2, b2, cw1, cb1, cw2, cb2, cw3, scl,
               mij_ref, ced_ref, cout_ref):
    cd = rr_ref[...] - rc_ref[...]
    radial = jnp.sum(cd * cd, axis=1, keepdims=True)
    if first:
        C = (radial <= CUTOFF).astype(jnp.float32)
        cout_ref[...] = C
    else:
        C = cin_ref[...]
    t = _sp(_mm(ea_ref[...], w1[...]) + b1[...])
    W = (_mm(t, w2[...]) + b2[...]) * C
    mij = xlg_ref[...] * W
    mij_ref[...] = mij
    nrm2 = jnp.sqrt(radial + 1e-8)
    cdn = cd / (nrm2 + 1.0)
    cn = jnp.sqrt(jnp.sum(cdn * cdn, axis=1, keepdims=True))
    cdn = cdn / jnp.maximum(cn, 1e-8) * scl[0, 0]
    t1 = jax.nn.silu(_mm(mij, cw1[...]) + cb1[...])
    t2 = jax.nn.silu(_mm(t1, cw2[...]) + cb2[...])
    ce = jnp.sum(t2 * cw3[...], axis=1, keepdims=True)
    ced_ref[...] = ce * cdn


def _edge_stage(first, ea, xlg, rr4, rc4, cin, w1, b1, w2, b2,
                cw1, cb1, cw2, cb2, cw3, scl):
    grid = E // BE
    especs = [
        pl.BlockSpec((BE, NG), lambda i: (i, 0)),
        pl.BlockSpec((BE, H), lambda i: (i, 0)),
        pl.BlockSpec((BE, 4), lambda i: (i, 0)),
        pl.BlockSpec((BE, 4), lambda i: (i, 0)),
        pl.BlockSpec((BE, 1), lambda i: (i, 0)),
    ]
    for w in (w1, b1, w2, b2, cw1, cb1, cw2, cb2, cw3, scl):
        especs.append(_full(w.shape))
    return pl.pallas_call(
        functools.partial(_edge_body, first),
        grid=(grid,),
        in_specs=especs,
        out_specs=[
            pl.BlockSpec((BE, H), lambda i: (i, 0)),
            pl.BlockSpec((BE, 4), lambda i: (i, 0)),
            pl.BlockSpec((BE, 1), lambda i: (i, 0)),
        ],
        out_shape=[
            jax.ShapeDtypeStruct((E, H), jnp.float32),
            jax.ShapeDtypeStruct((E, 4), jnp.float32),
            jax.ShapeDtypeStruct((E, 1), jnp.float32),
        ],
    )(ea, xlg, rr4, rc4, cin, w1, b1, w2, b2, cw1, cb1, cw2, cb2, cw3, scl)


def _node_body(h_ref, m0_ref, m1_ref, racc_ref, r4_ref,
               l2, l2b, lwa, lwb, lb, l1n,
               hn_ref, xln_ref, r4n_ref):
    m = _mm(m0_ref[...] + m1_ref[...], l2[...]) + l2b[...]
    hn = _mm(h_ref[...], lwa[...]) + _mm(_sp(m), lwb[...]) + lb[...]
    hn_ref[...] = hn
    xln_ref[...] = _mm(hn, l1n[...])
    rows = pl.program_id(0) * BN + jax.lax.broadcasted_iota(jnp.int32, (BN, 1), 0)
    upd = jnp.sum(racc_ref[...], axis=0)
    r4n_ref[...] = r4_ref[...] + jnp.where(rows < NL, upd, 0.0)


def _node_stage(h, m0, m1, racc, r4, l2, l2b, lwa, lwb, lb, l1n):
    grid = N // BN
    specs = [
        pl.BlockSpec((BN, H), lambda i: (i, 0)),
        pl.BlockSpec((BN, H), lambda i: (i, 0)),
        pl.BlockSpec((BN, H), lambda i: (i, 0)),
        pl.BlockSpec((2, BN, 4), lambda i: (0, i, 0)),
        pl.BlockSpec((BN, 4), lambda i: (i, 0)),
    ]
    for w in (l2, l2b, lwa, lwb, lb, l1n):
        specs.append(_full(w.shape))
    return pl.pallas_call(
        _node_body,
        grid=(grid,),
        in_specs=specs,
        out_specs=[
            pl.BlockSpec((BN, H), lambda i: (i, 0)),
            pl.BlockSpec((BN, H), lambda i: (i, 0)),
            pl.BlockSpec((BN, 4), lambda i: (i, 0)),
        ],
        out_shape=[
            jax.ShapeDtypeStruct((N, H), jnp.float32),
            jax.ShapeDtypeStruct((N, H), jnp.float32),
            jax.ShapeDtypeStruct((N, 4), jnp.float32),
        ],
    )(h, m0, m1, racc, r4, l2, l2b, lwa, lwb, lb, l1n)


def kernel(z, pos, edge_index, edge_attr, ligand_batch,
           le_w1, le_b1, le_w2, le_b2, pe_w1, pe_b1, pe_w2, pe_b2,
           mlp_w1, mlp_b1, mlp_w2, mlp_b2, lin1_w, lin2_w, lin2_b,
           lin_w, lin_b, cn_w1, cn_b1, cn_w2, cn_b2, cn_w3, scale):
    row = edge_index[0].astype(jnp.int32)
    col = edge_index[1].astype(jnp.int32)
    r4 = jnp.concatenate([pos, jnp.zeros((N, 1), jnp.float32)], axis=1)

    b2d = lambda b: b.reshape(1, -1)
    h, xl = _encoder(z, le_w1, b2d(le_b1), le_w2, b2d(le_b2),
                     pe_w1, b2d(pe_b1), pe_w2, b2d(pe_b2), lin1_w[0])

    zeros_h = jnp.zeros((N, H), jnp.float32)
    C = jnp.zeros((E, 1), jnp.float32)

    for i in range(L):
        xlg, rrflat, rcflat = _sc_gather(xl, r4.reshape(-1), row, col)
        rr4 = rrflat.reshape(E, 4)
        rc4 = rcflat.reshape(E, 4)
        mij, ced, cout = _edge_stage(
            i == 0, edge_attr, xlg, rr4, rc4, C,
            mlp_w1[i], b2d(mlp_b1[i]), mlp_w2[i], b2d(mlp_b2[i]),
            cn_w1[i], b2d(cn_b1[i]), cn_w2[i], b2d(cn_b2[i]),
            cn_w3[i], scale[i].reshape(1, 1))
        if i == 0:
            C = cout
        macc, = _sc_scatter(mij, col, zeros_h)
        racc, = _sc_rscatter(ced.reshape(-1), col, zeros_h)
        racc4 = racc.reshape(2, RROWS * H)[:, :N * 4].reshape(2, N, 4)
        l1n = lin1_w[(i + 1) % L]
        h, xl, r4 = _node_stage(
            h, macc[0], macc[1], racc4, r4,
            lin2_w[i], b2d(lin2_b[i]),
            lin_w[i][:, :H], lin_w[i][:, H:], b2d(lin_b[i]), l1n)

    return (h, r4[:, :3])


# speedup vs baseline: 2.8506x; 1.2380x over previous
import functools

import jax
import jax.numpy as jnp
from jax import lax
from jax.experimental import pallas as pl
from jax.experimental.pallas import tpu as pltpu
from jax.experimental.pallas import tpu_sc as plsc

H = 128
NG = 100
L = 6
IN_DIM = 5
CUTOFF = 10.0
N = 10000
E = 320000
NL = 2000

BN = 2000
BE = 2000

_LOG2 = 0.6931471805599453


def _mm(a, b):
    return jax.lax.dot_general(a, b, (((1,), (1,)), ((), ())),
                               preferred_element_type=jnp.float32)


def _sp(x):
    return jax.nn.softplus(x) - _LOG2


NWORK = 32
EPW = E // NWORK
CH = 80
NCH = EPW // CH
SLAB = 640
NSLAB_LAST = N - 15 * SLAB

_sc_mesh = plsc.VectorSubcoreMesh(core_axis_name="c", subcore_axis_name="s",
                                  num_cores=2, num_subcores=16)

CHG = 400
NCHG = EPW // CHG


def _sc_gather_body(xl_hbm, r4_hbm, row_hbm, col_hbm,
                    xlg_hbm, rr_hbm, rc_hbm,
                    ir0, ir1, ir2, ir3, ir4, idxc, xbuf, rrbuf, rcbuf,
                    r4loc, sem):
    cid = lax.axis_index("c")
    sid = lax.axis_index("s")
    base = (sid * 2 + cid) * EPW
    pltpu.sync_copy(r4_hbm, r4loc)
    iota = lax.iota(jnp.int32, 16)
    pos16 = iota // 4
    comp16 = iota % 4
    irs = (ir0, ir1, ir2, ir3, ir4)

    def chunk(j, _):
        off = pl.multiple_of(base + j * CHG, 8)
        for k in range(5):
            offk = pl.multiple_of(off + k * CH, 8)
            pltpu.sync_copy(row_hbm.at[pl.ds(offk, CH)], irs[k])
        pltpu.sync_copy(col_hbm.at[pl.ds(off, CHG)], idxc)
        descs = [pltpu.async_copy(xl_hbm.at[irs[k]],
                                  xbuf.at[pl.ds(k * CH, CH)], sem)
                 for k in range(5)]
        for k in range(5):
            for i in range(CH * 4 // 16):
                pv = i * 4 + pos16
                er = plsc.load_gather(irs[k], [pv])
                rrbuf[pl.ds(k * CH * 4 + i * 16, 16)] = plsc.load_gather(
                    r4loc, [er * 4 + comp16])
        for i in range(CHG * 4 // 16):
            pv = i * 4 + pos16
            ec = plsc.load_gather(idxc, [pv])
            rcbuf[pl.ds(i * 16, 16)] = plsc.load_gather(
                r4loc, [ec * 4 + comp16])
        off4 = pl.multiple_of(off * 4, 8)
        pltpu.sync_copy(rrbuf, rr_hbm.at[pl.ds(off4, CHG * 4)])
        pltpu.sync_copy(rcbuf, rc_hbm.at[pl.ds(off4, CHG * 4)])
        for d in descs:
            d.wait()
        pltpu.sync_copy(xbuf, xlg_hbm.at[pl.ds(off, CHG)])
        return _

    lax.fori_loop(0, NCHG, chunk, None)


def _sc_gather(xl, r4flat, row, col):
    f = pl.kernel(
        _sc_gather_body,
        out_type=[jax.ShapeDtypeStruct((E, H), jnp.float32),
                  jax.ShapeDtypeStruct((E * 4,), jnp.float32),
                  jax.ShapeDtypeStruct((E * 4,), jnp.float32)],
        mesh=_sc_mesh,
        scratch_types=[
            pltpu.VMEM((CH,), jnp.int32),
            pltpu.VMEM((CH,), jnp.int32),
            pltpu.VMEM((CH,), jnp.int32),
            pltpu.VMEM((CH,), jnp.int32),
            pltpu.VMEM((CH,), jnp.int32),
            pltpu.VMEM((CHG,), jnp.int32),
            pltpu.VMEM((CHG, H), jnp.float32),
            pltpu.VMEM((CHG * 4,), jnp.float32),
            pltpu.VMEM((CHG * 4,), jnp.float32),
            pltpu.VMEM((N * 4,), jnp.float32),
            pltpu.SemaphoreType.DMA,
        ],
        compiler_params=pltpu.CompilerParams(needs_layout_passes=False),
    )
    return f(xl, r4flat, row, col)


RROWS = 320


CHS = 320
NCHS = EPW // CHS


def _sc_scatter_body(mij_hbm, col_hbm, z128_hbm,
                     macc_hbm,
                     ic0, ic1, ic2, ic3, mbuf, accm, sem):
    cid = lax.axis_index("c")
    sid = lax.axis_index("s")
    base = (sid * 2 + cid) * EPW

    def slab_io(fn):
        @pl.when(sid < 15)
        def _():
            off = pl.multiple_of(sid * SLAB, 8)
            fn(pl.ds(off, SLAB))

        @pl.when(sid == 15)
        def _():
            fn(pl.ds(15 * SLAB, NSLAB_LAST))

    def init(slab):
        pltpu.sync_copy(z128_hbm.at[slab], accm.at[slab])

    slab_io(init)
    plsc.subcore_barrier()
    ics = (ic0, ic1, ic2, ic3)

    def do_group(off, nsub):
        pltpu.sync_copy(mij_hbm.at[pl.ds(off, nsub * CH)],
                        mbuf.at[pl.ds(0, nsub * CH)])
        for k in range(nsub):
            offk = pl.multiple_of(off + k * CH, 8)
            pltpu.sync_copy(col_hbm.at[pl.ds(offk, CH)], ics[k])
        descs = [pltpu.async_copy(mbuf.at[pl.ds(k * CH, CH)],
                                  accm.at[ics[k]], sem, add=True)
                 for k in range(nsub)]
        for d in descs:
            d.wait()

    def chunk(j, _):
        do_group(pl.multiple_of(base + j * CHS, 8), 4)
        return _

    lax.fori_loop(0, NCHS, chunk, None)
    do_group(pl.multiple_of(base + NCHS * CHS, 8), 1)
    plsc.subcore_barrier()

    def flush(slab):
        pltpu.sync_copy(accm.at[slab], macc_hbm.at[cid, slab])

    slab_io(flush)


def _sc_scatter(mij, col, z128):
    f = pl.kernel(
        _sc_scatter_body,
        out_type=[jax.ShapeDtypeStruct((2, N, H), jnp.float32)],
        mesh=_sc_mesh,
        scratch_types=[
            pltpu.VMEM((CH,), jnp.int32),
            pltpu.VMEM((CH,), jnp.int32),
            pltpu.VMEM((CH,), jnp.int32),
            pltpu.VMEM((CH,), jnp.int32),
            pltpu.VMEM((CHS, H), jnp.float32),
            pltpu.VMEM_SHARED((N, H), jnp.float32),
            pltpu.SemaphoreType.DMA,
        ],
    )
    return f(mij, col, z128)


CHR = 2000
NCHR = EPW // CHR


def _sc_rscatter_body(ced_hbm, col_hbm, z128_hbm,
                      racc_hbm,
                      cidx, cbuf, idxbuf, accr_sh, accr, sem):
    cid = lax.axis_index("c")
    sid = lax.axis_index("s")
    base = (sid * 2 + cid) * EPW
    iota = lax.iota(jnp.int32, 16)
    pos16 = iota // 4
    comp16 = iota % 4

    pltpu.sync_copy(z128_hbm.at[pl.ds(0, RROWS)], accr)

    @pl.when(sid == 0)
    def _():
        pltpu.sync_copy(z128_hbm.at[pl.ds(0, RROWS)], accr_sh)

    plsc.subcore_barrier()

    def chunk(j, _):
        off = pl.multiple_of(base + j * CHR, 8)
        pltpu.sync_copy(col_hbm.at[pl.ds(off, CHR)], cidx)
        off4 = pl.multiple_of(off * 4, 8)
        pltpu.sync_copy(ced_hbm.at[pl.ds(off4, CHR * 4)], cbuf)

        def group(i, _):
            pv = i * 4 + pos16
            ec = plsc.load_gather(cidx, [pv])
            a = ec * 4 + comp16
            plsc.addupdate_scatter(accr, [a // 128, a % 128],
                                   cbuf[pl.ds(i * 16, 16)])
            return _

        lax.fori_loop(0, CHR * 4 // 16, group, None)
        return _

    lax.fori_loop(0, NCHR, chunk, None)
    for c3 in range(RROWS // CH):
        for k in range(CH // 16):
            idxbuf[pl.ds(k * 16, 16)] = c3 * CH + k * 16 + iota
        pltpu.sync_copy(accr.at[pl.ds(c3 * CH, CH)], accr_sh.at[idxbuf],
                        add=True)
    plsc.subcore_barrier()

    @pl.when(sid < 2)
    def _():
        half = pl.multiple_of(sid * (RROWS // 2), 8)
        pltpu.sync_copy(accr_sh.at[pl.ds(half, RROWS // 2)],
                        racc_hbm.at[cid, pl.ds(half, RROWS // 2)])


def _sc_rscatter(cedflat, col, z128):
    f = pl.kernel(
        _sc_rscatter_body,
        out_type=[jax.ShapeDtypeStruct((2, RROWS, H), jnp.float32)],
        mesh=_sc_mesh,
        scratch_types=[
            pltpu.VMEM((CHR,), jnp.int32),
            pltpu.VMEM((CHR * 4,), jnp.float32),
            pltpu.VMEM((CH,), jnp.int32),
            pltpu.VMEM_SHARED((RROWS, H), jnp.float32),
            pltpu.VMEM((RROWS, H), jnp.float32),
            pltpu.SemaphoreType.DMA,
        ],
        compiler_params=pltpu.CompilerParams(needs_layout_passes=False),
    )
    return f(cedflat, col, z128)


def _enc_body(z_ref, lew1, leb1, lew2, leb2, pew1, peb1, pew2, peb2, l1,
              h_ref, xl_ref):
    z = z_ref[...]
    hl = _mm(jax.nn.silu(_mm(z, lew1[...]) + leb1[...]), lew2[...]) + leb2[...]
    hp = _mm(jax.nn.silu(_mm(z, pew1[...]) + peb1[...]), pew2[...]) + peb2[...]
    rows = pl.program_id(0) * BN + jax.lax.broadcasted_iota(jnp.int32, (BN, 1), 0)
    h = jnp.where(rows < NL, hl, hp)
    h_ref[...] = h
    xl_ref[...] = _mm(h, l1[...])


def _full(shape):
    nd = len(shape)
    return pl.BlockSpec(shape, lambda *_: (0,) * nd)


def _encoder(z, lew1, leb1, lew2, leb2, pew1, peb1, pew2, peb2, l1):
    grid = N // BN
    specs = [pl.BlockSpec((BN, IN_DIM), lambda i: (i, 0))]
    for w in (lew1, leb1, lew2, leb2, pew1, peb1, pew2, peb2, l1):
        specs.append(_full(w.shape))
    return pl.pallas_call(
        _enc_body,
        grid=(grid,),
        in_specs=specs,
        out_specs=[pl.BlockSpec((BN, H), lambda i: (i, 0))] * 2,
        out_shape=[jax.ShapeDtypeStruct((N, H), jnp.float32)] * 2,
    )(z, lew1, leb1, lew2, leb2, pew1, peb1, pew2, peb2, l1)


def _edge_body(first, ea_ref, xlg_ref, rr_ref, rc_ref, cin_ref,
               w1, b1, w2, b2, cw1, cb1, cw2, cb2, cw3, scl,
               mij_ref, ced_ref, cout_ref):
    cd = rr_ref[...] - rc_ref[...]
    radial = jnp.sum(cd * cd, axis=1, keepdims=True)
    if first:
        C = (radial <= CUTOFF).astype(jnp.float32)
        cout_ref[...] = C
    else:
        C = cin_ref[...]
    t = _sp(_mm(ea_ref[...], w1[...]) + b1[...])
    W = (_mm(t, w2[...]) + b2[...]) * C
    mij = xlg_ref[...] * W
    mij_ref[...] = mij
    nrm2 = jnp.sqrt(radial + 1e-8)
    cdn = cd / (nrm2 + 1.0)
    cn = jnp.sqrt(jnp.sum(cdn * cdn, axis=1, keepdims=True))
    cdn = cdn / jnp.maximum(cn, 1e-8) * scl[0, 0]
    t1 = jax.nn.silu(_mm(mij, cw1[...]) + cb1[...])
    t2 = jax.nn.silu(_mm(t1, cw2[...]) + cb2[...])
    ce = jnp.sum(t2 * cw3[...], axis=1, keepdims=True)
    ced_ref[...] = ce * cdn


def _edge_stage(first, ea, xlg, rr4, rc4, cin, w1, b1, w2, b2,
                cw1, cb1, cw2, cb2, cw3, scl):
    grid = E // BE
    especs = [
        pl.BlockSpec((BE, NG), lambda i: (i, 0)),
        pl.BlockSpec((BE, H), lambda i: (i, 0)),
        pl.BlockSpec((BE, 4), lambda i: (i, 0)),
        pl.BlockSpec((BE, 4), lambda i: (i, 0)),
        pl.BlockSpec((BE, 1), lambda i: (i, 0)),
    ]
    for w in (w1, b1, w2, b2, cw1, cb1, cw2, cb2, cw3, scl):
        especs.append(_full(w.shape))
    return pl.pallas_call(
        functools.partial(_edge_body, first),
        grid=(grid,),
        in_specs=especs,
        out_specs=[
            pl.BlockSpec((BE, H), lambda i: (i, 0)),
            pl.BlockSpec((BE, 4), lambda i: (i, 0)),
            pl.BlockSpec((BE, 1), lambda i: (i, 0)),
        ],
        out_shape=[
            jax.ShapeDtypeStruct((E, H), jnp.float32),
            jax.ShapeDtypeStruct((E, 4), jnp.float32),
            jax.ShapeDtypeStruct((E, 1), jnp.float32),
        ],
    )(ea, xlg, rr4, rc4, cin, w1, b1, w2, b2, cw1, cb1, cw2, cb2, cw3, scl)


def _node_body(h_ref, m0_ref, m1_ref, racc_ref, r4_ref,
               l2, l2b, lwa, lwb, lb, l1n,
               hn_ref, xln_ref, r4n_ref):
    m = _mm(m0_ref[...] + m1_ref[...], l2[...]) + l2b[...]
    hn = _mm(h_ref[...], lwa[...]) + _mm(_sp(m), lwb[...]) + lb[...]
    hn_ref[...] = hn
    xln_ref[...] = _mm(hn, l1n[...])
    rows = pl.program_id(0) * BN + jax.lax.broadcasted_iota(jnp.int32, (BN, 1), 0)
    upd = jnp.sum(racc_ref[...], axis=0)
    r4n_ref[...] = r4_ref[...] + jnp.where(rows < NL, upd, 0.0)


def _node_stage(h, m0, m1, racc, r4, l2, l2b, lwa, lwb, lb, l1n):
    grid = N // BN
    specs = [
        pl.BlockSpec((BN, H), lambda i: (i, 0)),
        pl.BlockSpec((BN, H), lambda i: (i, 0)),
        pl.BlockSpec((BN, H), lambda i: (i, 0)),
        pl.BlockSpec((2, BN, 4), lambda i: (0, i, 0)),
        pl.BlockSpec((BN, 4), lambda i: (i, 0)),
    ]
    for w in (l2, l2b, lwa, lwb, lb, l1n):
        specs.append(_full(w.shape))
    return pl.pallas_call(
        _node_body,
        grid=(grid,),
        in_specs=specs,
        out_specs=[
            pl.BlockSpec((BN, H), lambda i: (i, 0)),
            pl.BlockSpec((BN, H), lambda i: (i, 0)),
            pl.BlockSpec((BN, 4), lambda i: (i, 0)),
        ],
        out_shape=[
            jax.ShapeDtypeStruct((N, H), jnp.float32),
            jax.ShapeDtypeStruct((N, H), jnp.float32),
            jax.ShapeDtypeStruct((N, 4), jnp.float32),
        ],
    )(h, m0, m1, racc, r4, l2, l2b, lwa, lwb, lb, l1n)


def kernel(z, pos, edge_index, edge_attr, ligand_batch,
           le_w1, le_b1, le_w2, le_b2, pe_w1, pe_b1, pe_w2, pe_b2,
           mlp_w1, mlp_b1, mlp_w2, mlp_b2, lin1_w, lin2_w, lin2_b,
           lin_w, lin_b, cn_w1, cn_b1, cn_w2, cn_b2, cn_w3, scale):
    row = edge_index[0].astype(jnp.int32)
    col = edge_index[1].astype(jnp.int32)
    r4 = jnp.concatenate([pos, jnp.zeros((N, 1), jnp.float32)], axis=1)

    b2d = lambda b: b.reshape(1, -1)
    h, xl = _encoder(z, le_w1, b2d(le_b1), le_w2, b2d(le_b2),
                     pe_w1, b2d(pe_b1), pe_w2, b2d(pe_b2), lin1_w[0])

    zeros_h = jnp.zeros((N, H), jnp.float32)
    C = jnp.zeros((E, 1), jnp.float32)

    for i in range(L):
        xlg, rrflat, rcflat = _sc_gather(xl, r4.reshape(-1), row, col)
        rr4 = rrflat.reshape(E, 4)
        rc4 = rcflat.reshape(E, 4)
        mij, ced, cout = _edge_stage(
            i == 0, edge_attr, xlg, rr4, rc4, C,
            mlp_w1[i], b2d(mlp_b1[i]), mlp_w2[i], b2d(mlp_b2[i]),
            cn_w1[i], b2d(cn_b1[i]), cn_w2[i], b2d(cn_b2[i]),
            cn_w3[i], scale[i].reshape(1, 1))
        if i == 0:
            C = cout
        macc, = _sc_scatter(mij, col, zeros_h)
        racc, = _sc_rscatter(ced.reshape(-1), col, zeros_h)
        racc4 = racc.reshape(2, RROWS * H)[:, :N * 4].reshape(2, N, 4)
        l1n = lin1_w[(i + 1) % L]
        h, xl, r4 = _node_stage(
            h, macc[0], macc[1], racc4, r4,
            lin2_w[i], b2d(lin2_b[i]),
            lin_w[i][:, :H], lin_w[i][:, H:], b2d(lin_b[i]), l1n)

    return (h, r4[:, :3])


# speedup vs baseline: 2.9590x; 1.0380x over previous
import functools

import jax
import jax.numpy as jnp
from jax import lax
from jax.experimental import pallas as pl
from jax.experimental.pallas import tpu as pltpu
from jax.experimental.pallas import tpu_sc as plsc

H = 128
NG = 100
L = 6
IN_DIM = 5
CUTOFF = 10.0
N = 10000
E = 320000
NL = 2000

BN = 2000
BE = 2000

_LOG2 = 0.6931471805599453


def _mm(a, b):
    return jax.lax.dot_general(a, b, (((1,), (1,)), ((), ())),
                               preferred_element_type=jnp.float32)


def _sp(x):
    return jax.nn.softplus(x) - _LOG2


NWORK = 32
EPW = E // NWORK
CH = 80
NCH = EPW // CH
SLAB = 640
NSLAB_LAST = N - 15 * SLAB

_sc_mesh = plsc.VectorSubcoreMesh(core_axis_name="c", subcore_axis_name="s",
                                  num_cores=2, num_subcores=16)

CHG = 400
NCHG = EPW // CHG


def _sc_gather_body(xl_hbm, r4_hbm, row_hbm, col_hbm,
                    xlg_hbm, rr_hbm, rc_hbm,
                    ira0, ira1, ira2, ira3, ira4, idxca,
                    irb0, irb1, irb2, irb3, irb4, idxcb,
                    xbuf, rrbuf, rcbuf, r4loc,
                    isema, isemb, gsem, ws0, ws1, ws2, ws3, ws4):
    cid = lax.axis_index("c")
    sid = lax.axis_index("s")
    base = (sid * 2 + cid) * EPW
    pltpu.sync_copy(r4_hbm, r4loc)
    iota = lax.iota(jnp.int32, 16)
    pos16 = iota // 4
    comp16 = iota % 4
    idxsets = (((ira0, ira1, ira2, ira3, ira4), idxca, isema),
               ((irb0, irb1, irb2, irb3, irb4), idxcb, isemb))
    wsems = (ws0, ws1, ws2, ws3, ws4)

    def fire_idx(j, setsel):
        irs, idxc, isem = idxsets[setsel]
        off = pl.multiple_of(base + j * CHG, 8)
        for k in range(5):
            offk = pl.multiple_of(off + k * CH, 8)
            pltpu.async_copy(row_hbm.at[pl.ds(offk, CH)], irs[k], isem)
        pltpu.async_copy(col_hbm.at[pl.ds(off, CHG)], idxc, isem)

    def chunk(j, b, first, last):
        irs, idxc, isem = idxsets[b]
        off = pl.multiple_of(base + j * CHG, 8)
        if not last:
            fire_idx(j + 1, 1 - b)
        for k in range(5):
            pltpu.make_async_copy(row_hbm.at[pl.ds(0, CH)], irs[k],
                                  isem).wait()
        pltpu.make_async_copy(col_hbm.at[pl.ds(0, CHG)], idxc, isem).wait()
        if not first:
            for k in range(5):
                pltpu.make_async_copy(xlg_hbm.at[pl.ds(0, CH)],
                                      xbuf.at[pl.ds(k * CH, CH)],
                                      wsems[k]).wait()
        gdescs = [pltpu.async_copy(xl_hbm.at[irs[k]],
                                   xbuf.at[pl.ds(k * CH, CH)], gsem)
                  for k in range(5)]
        for k in range(5):
            for i in range(CH * 4 // 16):
                pv = i * 4 + pos16
                er = plsc.load_gather(irs[k], [pv])
                rrbuf[pl.ds(k * CH * 4 + i * 16, 16)] = plsc.load_gather(
                    r4loc, [er * 4 + comp16])
        for i in range(CHG * 4 // 16):
            pv = i * 4 + pos16
            ec = plsc.load_gather(idxc, [pv])
            rcbuf[pl.ds(i * 16, 16)] = plsc.load_gather(
                r4loc, [ec * 4 + comp16])
        off4 = pl.multiple_of(off * 4, 8)
        pltpu.sync_copy(rrbuf, rr_hbm.at[pl.ds(off4, CHG * 4)])
        pltpu.sync_copy(rcbuf, rc_hbm.at[pl.ds(off4, CHG * 4)])
        for d in gdescs:
            d.wait()
        for k in range(5):
            offk = pl.multiple_of(off + k * CH, 8)
            pltpu.async_copy(xbuf.at[pl.ds(k * CH, CH)],
                             xlg_hbm.at[pl.ds(offk, CH)], wsems[k])

    fire_idx(0, 0)
    chunk(0, 0, True, False)

    def pair(p, _):
        j = 1 + p * 2
        chunk(j, 1, False, False)
        chunk(j + 1, 0, False, False)
        return _

    lax.fori_loop(0, (NCHG - 3) // 2, pair, None)
    chunk(NCHG - 2, 1, False, False)
    chunk(NCHG - 1, 0, False, True)
    for k in range(5):
        pltpu.make_async_copy(xlg_hbm.at[pl.ds(0, CH)],
                              xbuf.at[pl.ds(k * CH, CH)], wsems[k]).wait()


def _sc_gather(xl, r4flat, row, col):
    f = pl.kernel(
        _sc_gather_body,
        out_type=[jax.ShapeDtypeStruct((E, H), jnp.float32),
                  jax.ShapeDtypeStruct((E * 4,), jnp.float32),
                  jax.ShapeDtypeStruct((E * 4,), jnp.float32)],
        mesh=_sc_mesh,
        scratch_types=(
            [pltpu.VMEM((CH,), jnp.int32)] * 5
            + [pltpu.VMEM((CHG,), jnp.int32)]
            + [pltpu.VMEM((CH,), jnp.int32)] * 5
            + [pltpu.VMEM((CHG,), jnp.int32)]
            + [
                pltpu.VMEM((CHG, H), jnp.float32),
                pltpu.VMEM((CHG * 4,), jnp.float32),
                pltpu.VMEM((CHG * 4,), jnp.float32),
                pltpu.VMEM((N * 4,), jnp.float32),
            ]
            + [pltpu.SemaphoreType.DMA] * 8
        ),
        compiler_params=pltpu.CompilerParams(needs_layout_passes=False),
    )
    return f(xl, r4flat, row, col)


RROWS = 320


CHS = 320
NCHS = EPW // CHS


def _sc_scatter_body(mij_hbm, col_hbm, z128_hbm,
                     macc_hbm,
                     ic0, ic1, ic2, ic3, mbuf, accm, sem):
    cid = lax.axis_index("c")
    sid = lax.axis_index("s")
    base = (sid * 2 + cid) * EPW

    def slab_io(fn):
        @pl.when(sid < 15)
        def _():
            off = pl.multiple_of(sid * SLAB, 8)
            fn(pl.ds(off, SLAB))

        @pl.when(sid == 15)
        def _():
            fn(pl.ds(15 * SLAB, NSLAB_LAST))

    def init(slab):
        pltpu.sync_copy(z128_hbm.at[slab], accm.at[slab])

    slab_io(init)
    plsc.subcore_barrier()
    ics = (ic0, ic1, ic2, ic3)

    def do_group(off, nsub):
        pltpu.sync_copy(mij_hbm.at[pl.ds(off, nsub * CH)],
                        mbuf.at[pl.ds(0, nsub * CH)])
        for k in range(nsub):
            offk = pl.multiple_of(off + k * CH, 8)
            pltpu.sync_copy(col_hbm.at[pl.ds(offk, CH)], ics[k])
        descs = [pltpu.async_copy(mbuf.at[pl.ds(k * CH, CH)],
                                  accm.at[ics[k]], sem, add=True)
                 for k in range(nsub)]
        for d in descs:
            d.wait()

    def chunk(j, _):
        do_group(pl.multiple_of(base + j * CHS, 8), 4)
        return _

    lax.fori_loop(0, NCHS, chunk, None)
    do_group(pl.multiple_of(base + NCHS * CHS, 8), 1)
    plsc.subcore_barrier()

    def flush(slab):
        pltpu.sync_copy(accm.at[slab], macc_hbm.at[cid, slab])

    slab_io(flush)


def _sc_scatter(mij, col, z128):
    f = pl.kernel(
        _sc_scatter_body,
        out_type=[jax.ShapeDtypeStruct((2, N, H), jnp.float32)],
        mesh=_sc_mesh,
        scratch_types=[
            pltpu.VMEM((CH,), jnp.int32),
            pltpu.VMEM((CH,), jnp.int32),
            pltpu.VMEM((CH,), jnp.int32),
            pltpu.VMEM((CH,), jnp.int32),
            pltpu.VMEM((CHS, H), jnp.float32),
            pltpu.VMEM_SHARED((N, H), jnp.float32),
            pltpu.SemaphoreType.DMA,
        ],
    )
    return f(mij, col, z128)


CHR = 2000
NCHR = EPW // CHR


def _sc_rscatter_body(ced_hbm, col_hbm, z128_hbm,
                      racc_hbm,
                      cidx, cbuf, idxbuf, accr_sh, accr, sem):
    cid = lax.axis_index("c")
    sid = lax.axis_index("s")
    base = (sid * 2 + cid) * EPW
    iota = lax.iota(jnp.int32, 16)
    pos16 = iota // 4
    comp16 = iota % 4

    pltpu.sync_copy(z128_hbm.at[pl.ds(0, RROWS)], accr)

    @pl.when(sid == 0)
    def _():
        pltpu.sync_copy(z128_hbm.at[pl.ds(0, RROWS)], accr_sh)

    plsc.subcore_barrier()

    def chunk(j, _):
        off = pl.multiple_of(base + j * CHR, 8)
        pltpu.sync_copy(col_hbm.at[pl.ds(off, CHR)], cidx)
        off4 = pl.multiple_of(off * 4, 8)
        pltpu.sync_copy(ced_hbm.at[pl.ds(off4, CHR * 4)], cbuf)

        def group(i, _):
            pv = i * 4 + pos16
            ec = plsc.load_gather(cidx, [pv])
            a = ec * 4 + comp16
            plsc.addupdate_scatter(accr, [a // 128, a % 128],
                                   cbuf[pl.ds(i * 16, 16)])
            return _

        lax.fori_loop(0, CHR * 4 // 16, group, None)
        return _

    lax.fori_loop(0, NCHR, chunk, None)
    for c3 in range(RROWS // CH):
        for k in range(CH // 16):
            idxbuf[pl.ds(k * 16, 16)] = c3 * CH + k * 16 + iota
        pltpu.sync_copy(accr.at[pl.ds(c3 * CH, CH)], accr_sh.at[idxbuf],
                        add=True)
    plsc.subcore_barrier()

    @pl.when(sid < 2)
    def _():
        half = pl.multiple_of(sid * (RROWS // 2), 8)
        pltpu.sync_copy(accr_sh.at[pl.ds(half, RROWS // 2)],
                        racc_hbm.at[cid, pl.ds(half, RROWS // 2)])


def _sc_rscatter(cedflat, col, z128):
    f = pl.kernel(
        _sc_rscatter_body,
        out_type=[jax.ShapeDtypeStruct((2, RROWS, H), jnp.float32)],
        mesh=_sc_mesh,
        scratch_types=[
            pltpu.VMEM((CHR,), jnp.int32),
            pltpu.VMEM((CHR * 4,), jnp.float32),
            pltpu.VMEM((CH,), jnp.int32),
            pltpu.VMEM_SHARED((RROWS, H), jnp.float32),
            pltpu.VMEM((RROWS, H), jnp.float32),
            pltpu.SemaphoreType.DMA,
        ],
        compiler_params=pltpu.CompilerParams(needs_layout_passes=False),
    )
    return f(cedflat, col, z128)


def _enc_body(z_ref, lew1, leb1, lew2, leb2, pew1, peb1, pew2, peb2, l1,
              h_ref, xl_ref):
    z = z_ref[...]
    hl = _mm(jax.nn.silu(_mm(z, lew1[...]) + leb1[...]), lew2[...]) + leb2[...]
    hp = _mm(jax.nn.silu(_mm(z, pew1[...]) + peb1[...]), pew2[...]) + peb2[...]
    rows = pl.program_id(0) * BN + jax.lax.broadcasted_iota(jnp.int32, (BN, 1), 0)
    h = jnp.where(rows < NL, hl, hp)
    h_ref[...] = h
    xl_ref[...] = _mm(h, l1[...])


def _full(shape):
    nd = len(shape)
    return pl.BlockSpec(shape, lambda *_: (0,) * nd)


def _encoder(z, lew1, leb1, lew2, leb2, pew1, peb1, pew2, peb2, l1):
    grid = N // BN
    specs = [pl.BlockSpec((BN, IN_DIM), lambda i: (i, 0))]
    for w in (lew1, leb1, lew2, leb2, pew1, peb1, pew2, peb2, l1):
        specs.append(_full(w.shape))
    return pl.pallas_call(
        _enc_body,
        grid=(grid,),
        in_specs=specs,
        out_specs=[pl.BlockSpec((BN, H), lambda i: (i, 0))] * 2,
        out_shape=[jax.ShapeDtypeStruct((N, H), jnp.float32)] * 2,
    )(z, lew1, leb1, lew2, leb2, pew1, peb1, pew2, peb2, l1)


def _edge_body(first, ea_ref, xlg_ref, rr_ref, rc_ref, cin_ref,
               w1, b1, w2, b2, cw1, cb1, cw2, cb2, cw3, scl,
               mij_ref, ced_ref, cout_ref):
    cd = rr_ref[...] - rc_ref[...]
    radial = jnp.sum(cd * cd, axis=1, keepdims=True)
    if first:
        C = (radial <= CUTOFF).astype(jnp.float32)
        cout_ref[...] = C
    else:
        C = cin_ref[...]
    t = _sp(_mm(ea_ref[...], w1[...]) + b1[...])
    W = (_mm(t, w2[...]) + b2[...]) * C
    mij = xlg_ref[...] * W
    mij_ref[...] = mij
    nrm2 = jnp.sqrt(radial + 1e-8)
    cdn = cd / (nrm2 + 1.0)
    cn = jnp.sqrt(jnp.sum(cdn * cdn, axis=1, keepdims=True))
    cdn = cdn / jnp.maximum(cn, 1e-8) * scl[0, 0]
    t1 = jax.nn.silu(_mm(mij, cw1[...]) + cb1[...])
    t2 = jax.nn.silu(_mm(t1, cw2[...]) + cb2[...])
    ce = jnp.sum(t2 * cw3[...], axis=1, keepdims=True)
    ced_ref[...] = ce * cdn


def _edge_stage(first, ea, xlg, rr4, rc4, cin, w1, b1, w2, b2,
                cw1, cb1, cw2, cb2, cw3, scl):
    grid = E // BE
    especs = [
        pl.BlockSpec((BE, NG), lambda i: (i, 0)),
        pl.BlockSpec((BE, H), lambda i: (i, 0)),
        pl.BlockSpec((BE, 4), lambda i: (i, 0)),
        pl.BlockSpec((BE, 4), lambda i: (i, 0)),
        pl.BlockSpec((BE, 1), lambda i: (i, 0)),
    ]
    for w in (w1, b1, w2, b2, cw1, cb1, cw2, cb2, cw3, scl):
        especs.append(_full(w.shape))
    return pl.pallas_call(
        functools.partial(_edge_body, first),
        grid=(grid,),
        in_specs=especs,
        out_specs=[
            pl.BlockSpec((BE, H), lambda i: (i, 0)),
            pl.BlockSpec((BE, 4), lambda i: (i, 0)),
            pl.BlockSpec((BE, 1), lambda i: (i, 0)),
        ],
        out_shape=[
            jax.ShapeDtypeStruct((E, H), jnp.float32),
            jax.ShapeDtypeStruct((E, 4), jnp.float32),
            jax.ShapeDtypeStruct((E, 1), jnp.float32),
        ],
    )(ea, xlg, rr4, rc4, cin, w1, b1, w2, b2, cw1, cb1, cw2, cb2, cw3, scl)


def _node_body(h_ref, m0_ref, m1_ref, racc_ref, r4_ref,
               l2, l2b, lwa, lwb, lb, l1n,
               hn_ref, xln_ref, r4n_ref):
    m = _mm(m0_ref[...] + m1_ref[...], l2[...]) + l2b[...]
    hn = _mm(h_ref[...], lwa[...]) + _mm(_sp(m), lwb[...]) + lb[...]
    hn_ref[...] = hn
    xln_ref[...] = _mm(hn, l1n[...])
    rows = pl.program_id(0) * BN + jax.lax.broadcasted_iota(jnp.int32, (BN, 1), 0)
    upd = jnp.sum(racc_ref[...], axis=0)
    r4n_ref[...] = r4_ref[...] + jnp.where(rows < NL, upd, 0.0)


def _node_stage(h, m0, m1, racc, r4, l2, l2b, lwa, lwb, lb, l1n):
    grid = N // BN
    specs = [
        pl.BlockSpec((BN, H), lambda i: (i, 0)),
        pl.BlockSpec((BN, H), lambda i: (i, 0)),
        pl.BlockSpec((BN, H), lambda i: (i, 0)),
        pl.BlockSpec((2, BN, 4), lambda i: (0, i, 0)),
        pl.BlockSpec((BN, 4), lambda i: (i, 0)),
    ]
    for w in (l2, l2b, lwa, lwb, lb, l1n):
        specs.append(_full(w.shape))
    return pl.pallas_call(
        _node_body,
        grid=(grid,),
        in_specs=specs,
        out_specs=[
            pl.BlockSpec((BN, H), lambda i: (i, 0)),
            pl.BlockSpec((BN, H), lambda i: (i, 0)),
            pl.BlockSpec((BN, 4), lambda i: (i, 0)),
        ],
        out_shape=[
            jax.ShapeDtypeStruct((N, H), jnp.float32),
            jax.ShapeDtypeStruct((N, H), jnp.float32),
            jax.ShapeDtypeStruct((N, 4), jnp.float32),
        ],
    )(h, m0, m1, racc, r4, l2, l2b, lwa, lwb, lb, l1n)


def kernel(z, pos, edge_index, edge_attr, ligand_batch,
           le_w1, le_b1, le_w2, le_b2, pe_w1, pe_b1, pe_w2, pe_b2,
           mlp_w1, mlp_b1, mlp_w2, mlp_b2, lin1_w, lin2_w, lin2_b,
           lin_w, lin_b, cn_w1, cn_b1, cn_w2, cn_b2, cn_w3, scale):
    row = edge_index[0].astype(jnp.int32)
    col = edge_index[1].astype(jnp.int32)
    r4 = jnp.concatenate([pos, jnp.zeros((N, 1), jnp.float32)], axis=1)

    b2d = lambda b: b.reshape(1, -1)
    h, xl = _encoder(z, le_w1, b2d(le_b1), le_w2, b2d(le_b2),
                     pe_w1, b2d(pe_b1), pe_w2, b2d(pe_b2), lin1_w[0])

    zeros_h = jnp.zeros((N, H), jnp.float32)
    C = jnp.zeros((E, 1), jnp.float32)

    for i in range(L):
        xlg, rrflat, rcflat = _sc_gather(xl, r4.reshape(-1), row, col)
        rr4 = rrflat.reshape(E, 4)
        rc4 = rcflat.reshape(E, 4)
        mij, ced, cout = _edge_stage(
            i == 0, edge_attr, xlg, rr4, rc4, C,
            mlp_w1[i], b2d(mlp_b1[i]), mlp_w2[i], b2d(mlp_b2[i]),
            cn_w1[i], b2d(cn_b1[i]), cn_w2[i], b2d(cn_b2[i]),
            cn_w3[i], scale[i].reshape(1, 1))
        if i == 0:
            C = cout
        macc, = _sc_scatter(mij, col, zeros_h)
        racc, = _sc_rscatter(ced.reshape(-1), col, zeros_h)
        racc4 = racc.reshape(2, RROWS * H)[:, :N * 4].reshape(2, N, 4)
        l1n = lin1_w[(i + 1) % L]
        h, xl, r4 = _node_stage(
            h, macc[0], macc[1], racc4, r4,
            lin2_w[i], b2d(lin2_b[i]),
            lin_w[i][:, :H], lin_w[i][:, H:], b2d(lin_b[i]), l1n)

    return (h, r4[:, :3])


# speedup vs baseline: 3.0573x; 1.0332x over previous
import functools

import jax
import jax.numpy as jnp
from jax import lax
from jax.experimental import pallas as pl
from jax.experimental.pallas import tpu as pltpu
from jax.experimental.pallas import tpu_sc as plsc

H = 128
NG = 100
L = 6
IN_DIM = 5
CUTOFF = 10.0
N = 10000
E = 320000
NL = 2000

BN = 2000
BE = 2000

_LOG2 = 0.6931471805599453


def _mm(a, b):
    return jax.lax.dot_general(a, b, (((1,), (1,)), ((), ())),
                               preferred_element_type=jnp.float32)


def _sp(x):
    return jax.nn.softplus(x) - _LOG2


NWORK = 32
EPW = E // NWORK
CH = 80
NCH = EPW // CH
SLAB = 640
NSLAB_LAST = N - 15 * SLAB

_sc_mesh = plsc.VectorSubcoreMesh(core_axis_name="c", subcore_axis_name="s",
                                  num_cores=2, num_subcores=16)

CHG = 400
NCHG = EPW // CHG


def _sc_gather_body(xl_hbm, r4_hbm, row_hbm, col_hbm,
                    xlg_hbm, rr_hbm, rc_hbm,
                    ira0, ira1, ira2, ira3, ira4, idxca,
                    irb0, irb1, irb2, irb3, irb4, idxcb,
                    xbuf, rrbuf, rcbuf, r4loc,
                    isema, isemb, gsem, ws0, ws1, ws2, ws3, ws4):
    cid = lax.axis_index("c")
    sid = lax.axis_index("s")
    base = (sid * 2 + cid) * EPW
    pltpu.sync_copy(r4_hbm, r4loc)
    iota = lax.iota(jnp.int32, 16)
    pos16 = iota // 4
    comp16 = iota % 4
    idxsets = (((ira0, ira1, ira2, ira3, ira4), idxca, isema),
               ((irb0, irb1, irb2, irb3, irb4), idxcb, isemb))
    wsems = (ws0, ws1, ws2, ws3, ws4)

    def fire_idx(j, setsel):
        irs, idxc, isem = idxsets[setsel]
        off = pl.multiple_of(base + j * CHG, 8)
        for k in range(5):
            offk = pl.multiple_of(off + k * CH, 8)
            pltpu.async_copy(row_hbm.at[pl.ds(offk, CH)], irs[k], isem)
        pltpu.async_copy(col_hbm.at[pl.ds(off, CHG)], idxc, isem)

    def chunk(j, b, first, last):
        irs, idxc, isem = idxsets[b]
        off = pl.multiple_of(base + j * CHG, 8)
        if not last:
            fire_idx(j + 1, 1 - b)
        for k in range(5):
            pltpu.make_async_copy(row_hbm.at[pl.ds(0, CH)], irs[k],
                                  isem).wait()
        pltpu.make_async_copy(col_hbm.at[pl.ds(0, CHG)], idxc, isem).wait()
        if not first:
            for k in range(5):
                pltpu.make_async_copy(xlg_hbm.at[pl.ds(0, CH)],
                                      xbuf.at[pl.ds(k * CH, CH)],
                                      wsems[k]).wait()
        gdescs = [pltpu.async_copy(xl_hbm.at[irs[k]],
                                   xbuf.at[pl.ds(k * CH, CH)], gsem)
                  for k in range(5)]
        for k in range(5):
            for i in range(CH * 4 // 16):
                pv = i * 4 + pos16
                er = plsc.load_gather(irs[k], [pv])
                rrbuf[pl.ds(k * CH * 4 + i * 16, 16)] = plsc.load_gather(
                    r4loc, [er * 4 + comp16])
        for i in range(CHG * 4 // 16):
            pv = i * 4 + pos16
            ec = plsc.load_gather(idxc, [pv])
            rcbuf[pl.ds(i * 16, 16)] = plsc.load_gather(
                r4loc, [ec * 4 + comp16])
        off4 = pl.multiple_of(off * 4, 8)
        pltpu.sync_copy(rrbuf, rr_hbm.at[pl.ds(off4, CHG * 4)])
        pltpu.sync_copy(rcbuf, rc_hbm.at[pl.ds(off4, CHG * 4)])
        for d in gdescs:
            d.wait()
        for k in range(5):
            offk = pl.multiple_of(off + k * CH, 8)
            pltpu.async_copy(xbuf.at[pl.ds(k * CH, CH)],
                             xlg_hbm.at[pl.ds(offk, CH)], wsems[k])

    fire_idx(0, 0)
    chunk(0, 0, True, False)

    def pair(p, _):
        j = 1 + p * 2
        chunk(j, 1, False, False)
        chunk(j + 1, 0, False, False)
        return _

    lax.fori_loop(0, (NCHG - 3) // 2, pair, None)
    chunk(NCHG - 2, 1, False, False)
    chunk(NCHG - 1, 0, False, True)
    for k in range(5):
        pltpu.make_async_copy(xlg_hbm.at[pl.ds(0, CH)],
                              xbuf.at[pl.ds(k * CH, CH)], wsems[k]).wait()


def _sc_gather(xl, r4flat, row, col):
    f = pl.kernel(
        _sc_gather_body,
        out_type=[jax.ShapeDtypeStruct((E, H), jnp.float32),
                  jax.ShapeDtypeStruct((E * 4,), jnp.float32),
                  jax.ShapeDtypeStruct((E * 4,), jnp.float32)],
        mesh=_sc_mesh,
        scratch_types=(
            [pltpu.VMEM((CH,), jnp.int32)] * 5
            + [pltpu.VMEM((CHG,), jnp.int32)]
            + [pltpu.VMEM((CH,), jnp.int32)] * 5
            + [pltpu.VMEM((CHG,), jnp.int32)]
            + [
                pltpu.VMEM((CHG, H), jnp.float32),
                pltpu.VMEM((CHG * 4,), jnp.float32),
                pltpu.VMEM((CHG * 4,), jnp.float32),
                pltpu.VMEM((N * 4,), jnp.float32),
            ]
            + [pltpu.SemaphoreType.DMA] * 8
        ),
        compiler_params=pltpu.CompilerParams(needs_layout_passes=False),
    )
    return f(xl, r4flat, row, col)


RROWS = 320


CHS = 320
NCHS = EPW // CHS


def _sc_scatter_body(mij_hbm, col_hbm, z128_hbm,
                     macc_hbm,
                     ic0, ic1, ic2, ic3, mbuf, accm, sem, isem):
    cid = lax.axis_index("c")
    sid = lax.axis_index("s")
    base = (sid * 2 + cid) * EPW

    def slab_io(fn):
        @pl.when(sid < 15)
        def _():
            off = pl.multiple_of(sid * SLAB, 8)
            fn(pl.ds(off, SLAB))

        @pl.when(sid == 15)
        def _():
            fn(pl.ds(15 * SLAB, NSLAB_LAST))

    def init(slab):
        pltpu.sync_copy(z128_hbm.at[slab], accm.at[slab])

    slab_io(init)
    plsc.subcore_barrier()
    ics = (ic0, ic1, ic2, ic3)

    def do_group(off, nsub):
        idescs = [pltpu.async_copy(
                      col_hbm.at[pl.ds(pl.multiple_of(off + k * CH, 8), CH)],
                      ics[k], isem)
                  for k in range(nsub)]
        pltpu.sync_copy(mij_hbm.at[pl.ds(off, nsub * CH)],
                        mbuf.at[pl.ds(0, nsub * CH)])
        for d in idescs:
            d.wait()
        descs = [pltpu.async_copy(mbuf.at[pl.ds(k * CH, CH)],
                                  accm.at[ics[k]], sem, add=True)
                 for k in range(nsub)]
        for d in descs:
            d.wait()

    def chunk(j, _):
        do_group(pl.multiple_of(base + j * CHS, 8), 4)
        return _

    lax.fori_loop(0, NCHS, chunk, None)
    do_group(pl.multiple_of(base + NCHS * CHS, 8), 1)
    plsc.subcore_barrier()

    def flush(slab):
        pltpu.sync_copy(accm.at[slab], macc_hbm.at[cid, slab])

    slab_io(flush)


def _sc_scatter(mij, col, z128):
    f = pl.kernel(
        _sc_scatter_body,
        out_type=[jax.ShapeDtypeStruct((2, N, H), jnp.float32)],
        mesh=_sc_mesh,
        scratch_types=[
            pltpu.VMEM((CH,), jnp.int32),
            pltpu.VMEM((CH,), jnp.int32),
            pltpu.VMEM((CH,), jnp.int32),
            pltpu.VMEM((CH,), jnp.int32),
            pltpu.VMEM((CHS, H), jnp.float32),
            pltpu.VMEM_SHARED((N, H), jnp.float32),
            pltpu.SemaphoreType.DMA,
            pltpu.SemaphoreType.DMA,
        ],
    )
    return f(mij, col, z128)


CHR = 2000
NCHR = EPW // CHR


def _sc_rscatter_body(ced_hbm, col_hbm, z128_hbm,
                      racc_hbm,
                      cidx, cbuf, idxbuf, accr_sh, accr, sem):
    cid = lax.axis_index("c")
    sid = lax.axis_index("s")
    base = (sid * 2 + cid) * EPW
    iota = lax.iota(jnp.int32, 16)
    pos16 = iota // 4
    comp16 = iota % 4

    pltpu.sync_copy(z128_hbm.at[pl.ds(0, RROWS)], accr)

    @pl.when(sid == 0)
    def _():
        pltpu.sync_copy(z128_hbm.at[pl.ds(0, RROWS)], accr_sh)

    plsc.subcore_barrier()

    def chunk(j, _):
        off = pl.multiple_of(base + j * CHR, 8)
        pltpu.sync_copy(col_hbm.at[pl.ds(off, CHR)], cidx)
        off4 = pl.multiple_of(off * 4, 8)
        pltpu.sync_copy(ced_hbm.at[pl.ds(off4, CHR * 4)], cbuf)

        def group(i, _):
            pv = i * 4 + pos16
            ec = plsc.load_gather(cidx, [pv])
            a = ec * 4 + comp16
            plsc.addupdate_scatter(accr, [a // 128, a % 128],
                                   cbuf[pl.ds(i * 16, 16)])
            return _

        lax.fori_loop(0, CHR * 4 // 16, group, None)
        return _

    lax.fori_loop(0, NCHR, chunk, None)
    for c3 in range(RROWS // CH):
        for k in range(CH // 16):
            idxbuf[pl.ds(k * 16, 16)] = c3 * CH + k * 16 + iota
        pltpu.sync_copy(accr.at[pl.ds(c3 * CH, CH)], accr_sh.at[idxbuf],
                        add=True)
    plsc.subcore_barrier()

    @pl.when(sid < 2)
    def _():
        half = pl.multiple_of(sid * (RROWS // 2), 8)
        pltpu.sync_copy(accr_sh.at[pl.ds(half, RROWS // 2)],
                        racc_hbm.at[cid, pl.ds(half, RROWS // 2)])


def _sc_rscatter(cedflat, col, z128):
    f = pl.kernel(
        _sc_rscatter_body,
        out_type=[jax.ShapeDtypeStruct((2, RROWS, H), jnp.float32)],
        mesh=_sc_mesh,
        scratch_types=[
            pltpu.VMEM((CHR,), jnp.int32),
            pltpu.VMEM((CHR * 4,), jnp.float32),
            pltpu.VMEM((CH,), jnp.int32),
            pltpu.VMEM_SHARED((RROWS, H), jnp.float32),
            pltpu.VMEM((RROWS, H), jnp.float32),
            pltpu.SemaphoreType.DMA,
        ],
        compiler_params=pltpu.CompilerParams(needs_layout_passes=False),
    )
    return f(cedflat, col, z128)


def _enc_body(z_ref, lew1, leb1, lew2, leb2, pew1, peb1, pew2, peb2, l1,
              h_ref, xl_ref):
    z = z_ref[...]
    hl = _mm(jax.nn.silu(_mm(z, lew1[...]) + leb1[...]), lew2[...]) + leb2[...]
    hp = _mm(jax.nn.silu(_mm(z, pew1[...]) + peb1[...]), pew2[...]) + peb2[...]
    rows = pl.program_id(0) * BN + jax.lax.broadcasted_iota(jnp.int32, (BN, 1), 0)
    h = jnp.where(rows < NL, hl, hp)
    h_ref[...] = h
    xl_ref[...] = _mm(h, l1[...])


def _full(shape):
    nd = len(shape)
    return pl.BlockSpec(shape, lambda *_: (0,) * nd)


def _encoder(z, lew1, leb1, lew2, leb2, pew1, peb1, pew2, peb2, l1):
    grid = N // BN
    specs = [pl.BlockSpec((BN, IN_DIM), lambda i: (i, 0))]
    for w in (lew1, leb1, lew2, leb2, pew1, peb1, pew2, peb2, l1):
        specs.append(_full(w.shape))
    return pl.pallas_call(
        _enc_body,
        grid=(grid,),
        in_specs=specs,
        out_specs=[pl.BlockSpec((BN, H), lambda i: (i, 0))] * 2,
        out_shape=[jax.ShapeDtypeStruct((N, H), jnp.float32)] * 2,
    )(z, lew1, leb1, lew2, leb2, pew1, peb1, pew2, peb2, l1)


def _edge_body(first, ea_ref, xlg_ref, rr_ref, rc_ref, cin_ref,
               w1, b1, w2, b2, cw1, cb1, cw2, cb2, cw3, scl,
               mij_ref, ced_ref, cout_ref):
    cd = rr_ref[...] - rc_ref[...]
    radial = jnp.sum(cd * cd, axis=1, keepdims=True)
    if first:
        C = (radial <= CUTOFF).astype(jnp.float32)
        cout_ref[...] = C
    else:
        C = cin_ref[...]
    t = _sp(_mm(ea_ref[...], w1[...]) + b1[...])
    W = (_mm(t, w2[...]) + b2[...]) * C
    mij = xlg_ref[...] * W
    mij_ref[...] = mij
    nrm2 = jnp.sqrt(radial + 1e-8)
    cdn = cd / (nrm2 + 1.0)
    cn = jnp.sqrt(jnp.sum(cdn * cdn, axis=1, keepdims=True))
    cdn = cdn / jnp.maximum(cn, 1e-8) * scl[0, 0]
    t1 = jax.nn.silu(_mm(mij, cw1[...]) + cb1[...])
    t2 = jax.nn.silu(_mm(t1, cw2[...]) + cb2[...])
    ce = jnp.sum(t2 * cw3[...], axis=1, keepdims=True)
    ced_ref[...] = ce * cdn


def _edge_stage(first, ea, xlg, rr4, rc4, cin, w1, b1, w2, b2,
                cw1, cb1, cw2, cb2, cw3, scl):
    grid = E // BE
    especs = [
        pl.BlockSpec((BE, NG), lambda i: (i, 0)),
        pl.BlockSpec((BE, H), lambda i: (i, 0)),
        pl.BlockSpec((BE, 4), lambda i: (i, 0)),
        pl.BlockSpec((BE, 4), lambda i: (i, 0)),
        pl.BlockSpec((BE, 1), lambda i: (i, 0)),
    ]
    for w in (w1, b1, w2, b2, cw1, cb1, cw2, cb2, cw3, scl):
        especs.append(_full(w.shape))
    return pl.pallas_call(
        functools.partial(_edge_body, first),
        grid=(grid,),
        in_specs=especs,
        out_specs=[
            pl.BlockSpec((BE, H), lambda i: (i, 0)),
            pl.BlockSpec((BE, 4), lambda i: (i, 0)),
            pl.BlockSpec((BE, 1), lambda i: (i, 0)),
        ],
        out_shape=[
            jax.ShapeDtypeStruct((E, H), jnp.float32),
            jax.ShapeDtypeStruct((E, 4), jnp.float32),
            jax.ShapeDtypeStruct((E, 1), jnp.float32),
        ],
    )(ea, xlg, rr4, rc4, cin, w1, b1, w2, b2, cw1, cb1, cw2, cb2, cw3, scl)


def _node_body(h_ref, m0_ref, m1_ref, racc_ref, r4_ref,
               l2, l2b, lwa, lwb, lb, l1n,
               hn_ref, xln_ref, r4n_ref):
    m = _mm(m0_ref[...] + m1_ref[...], l2[...]) + l2b[...]
    hn = _mm(h_ref[...], lwa[...]) + _mm(_sp(m), lwb[...]) + lb[...]
    hn_ref[...] = hn
    xln_ref[...] = _mm(hn, l1n[...])
    rows = pl.program_id(0) * BN + jax.lax.broadcasted_iota(jnp.int32, (BN, 1), 0)
    upd = jnp.sum(racc_ref[...], axis=0)
    r4n_ref[...] = r4_ref[...] + jnp.where(rows < NL, upd, 0.0)


def _node_stage(h, m0, m1, racc, r4, l2, l2b, lwa, lwb, lb, l1n):
    grid = N // BN
    specs = [
        pl.BlockSpec((BN, H), lambda i: (i, 0)),
        pl.BlockSpec((BN, H), lambda i: (i, 0)),
        pl.BlockSpec((BN, H), lambda i: (i, 0)),
        pl.BlockSpec((2, BN, 4), lambda i: (0, i, 0)),
        pl.BlockSpec((BN, 4), lambda i: (i, 0)),
    ]
    for w in (l2, l2b, lwa, lwb, lb, l1n):
        specs.append(_full(w.shape))
    return pl.pallas_call(
        _node_body,
        grid=(grid,),
        in_specs=specs,
        out_specs=[
            pl.BlockSpec((BN, H), lambda i: (i, 0)),
            pl.BlockSpec((BN, H), lambda i: (i, 0)),
            pl.BlockSpec((BN, 4), lambda i: (i, 0)),
        ],
        out_shape=[
            jax.ShapeDtypeStruct((N, H), jnp.float32),
            jax.ShapeDtypeStruct((N, H), jnp.float32),
            jax.ShapeDtypeStruct((N, 4), jnp.float32),
        ],
    )(h, m0, m1, racc, r4, l2, l2b, lwa, lwb, lb, l1n)


def kernel(z, pos, edge_index, edge_attr, ligand_batch,
           le_w1, le_b1, le_w2, le_b2, pe_w1, pe_b1, pe_w2, pe_b2,
           mlp_w1, mlp_b1, mlp_w2, mlp_b2, lin1_w, lin2_w, lin2_b,
           lin_w, lin_b, cn_w1, cn_b1, cn_w2, cn_b2, cn_w3, scale):
    row = edge_index[0].astype(jnp.int32)
    col = edge_index[1].astype(jnp.int32)
    r4 = jnp.concatenate([pos, jnp.zeros((N, 1), jnp.float32)], axis=1)

    b2d = lambda b: b.reshape(1, -1)
    h, xl = _encoder(z, le_w1, b2d(le_b1), le_w2, b2d(le_b2),
                     pe_w1, b2d(pe_b1), pe_w2, b2d(pe_b2), lin1_w[0])

    zeros_h = jnp.zeros((N, H), jnp.float32)
    C = jnp.zeros((E, 1), jnp.float32)

    for i in range(L):
        xlg, rrflat, rcflat = _sc_gather(xl, r4.reshape(-1), row, col)
        rr4 = rrflat.reshape(E, 4)
        rc4 = rcflat.reshape(E, 4)
        mij, ced, cout = _edge_stage(
            i == 0, edge_attr, xlg, rr4, rc4, C,
            mlp_w1[i], b2d(mlp_b1[i]), mlp_w2[i], b2d(mlp_b2[i]),
            cn_w1[i], b2d(cn_b1[i]), cn_w2[i], b2d(cn_b2[i]),
            cn_w3[i], scale[i].reshape(1, 1))
        if i == 0:
            C = cout
        macc, = _sc_scatter(mij, col, zeros_h)
        racc, = _sc_rscatter(ced.reshape(-1), col, zeros_h)
        racc4 = racc.reshape(2, RROWS * H)[:, :N * 4].reshape(2, N, 4)
        l1n = lin1_w[(i + 1) % L]
        h, xl, r4 = _node_stage(
            h, macc[0], macc[1], racc4, r4,
            lin2_w[i], b2d(lin2_b[i]),
            lin_w[i][:, :H], lin_w[i][:, H:], b2d(lin_b[i]), l1n)

    return (h, r4[:, :3])
